# Initial kernel scaffold; baseline (speedup 1.0000x reference)
import functools

import jax
import jax.numpy as jnp
from jax import lax
from jax.experimental import pallas as pl
from jax.experimental.pallas import tpu as pltpu
from jax.experimental.pallas import tpu_sc as plsc

N = 10000
E = 320000
NC, NS = 2, 16
NW = NC * NS
CHUNK = 128
CPW = 80
EP = NW * CHUNK * CPW
N2 = 10240
RPT = N2 // NS
BLK = 2048
NBLK = EP // BLK


def _sc_mesh():
    return plsc.VectorSubcoreMesh(core_axis_name="c", subcore_axis_name="s",
                                  num_cores=NC, num_subcores=NS)


def _make_gather(WA, WB):
    @functools.partial(
        pl.kernel,
        out_type=(jax.ShapeDtypeStruct((EP, WA), jnp.float32),
                  jax.ShapeDtypeStruct((EP, WB), jnp.float32)),
        mesh=_sc_mesh(),
        scratch_types=[
            pltpu.VMEM((CHUNK,), jnp.int32),
            pltpu.VMEM((CHUNK,), jnp.int32),
            pltpu.VMEM((CHUNK, WA), jnp.float32),
            pltpu.VMEM((CHUNK, WB), jnp.float32),
            pltpu.SemaphoreType.DMA,
            pltpu.SemaphoreType.DMA,
        ],
    )
    def k(ta, tb, ia_hbm, ib_hbm, oa, ob, ia_v, ib_v, ra_v, rb_v, sa, sb):
        cid = lax.axis_index("c")
        sid = lax.axis_index("s")
        wid = sid * NC + cid
        base_w = wid * (CPW * CHUNK)

        def body(i, carry):
            base = base_w + i * CHUNK
            pltpu.sync_copy(ia_hbm.at[pl.ds(base, CHUNK)], ia_v)
            pltpu.sync_copy(ib_hbm.at[pl.ds(base, CHUNK)], ib_v)
            ca = pltpu.async_copy(ta.at[ia_v], ra_v, sa)
            cb = pltpu.async_copy(tb.at[ib_v], rb_v, sb)
            ca.wait()
            cb.wait()
            pltpu.sync_copy(ra_v, oa.at[pl.ds(base, CHUNK)])
            pltpu.sync_copy(rb_v, ob.at[pl.ds(base, CHUNK)])
            return carry

        lax.fori_loop(0, CPW, body, 0)

    return k


def _make_scatter(W):
    @functools.partial(
        pl.kernel,
        out_type=jax.ShapeDtypeStruct((NC * N2, W), jnp.float32),
        mesh=_sc_mesh(),
        scratch_types=[
            pltpu.VMEM((CHUNK,), jnp.int32),
            pltpu.VMEM((CHUNK, W), jnp.float32),
            pltpu.VMEM_SHARED((N2, W), jnp.float32),
            pltpu.SemaphoreType.DMA,
        ],
    )
    def k(pay_hbm, dst_hbm, zeros_hbm, out, idx_v, pay_v, accum, sem):
        cid = lax.axis_index("c")
        sid = lax.axis_index("s")
        wid = sid * NC + cid
        rbase = sid * RPT
        pltpu.sync_copy(zeros_hbm.at[pl.ds(rbase, RPT)],
                        accum.at[pl.ds(rbase, RPT)])
        plsc.subcore_barrier()
        base_w = wid * (CPW * CHUNK)
        nreal = jnp.minimum(CPW, (E - base_w) // CHUNK)

        def body(i, carry):
            base = base_w + i * CHUNK
            pltpu.sync_copy(dst_hbm.at[pl.ds(base, CHUNK)], idx_v)
            pltpu.sync_copy(pay_hbm.at[pl.ds(base, CHUNK)], pay_v)
            pltpu.sync_copy(pay_v, accum.at[idx_v], add=True)
            return carry

        lax.fori_loop(0, nreal, body, 0)
        plsc.subcore_barrier()
        pltpu.sync_copy(accum.at[pl.ds(rbase, RPT)],
                        out.at[pl.ds(cid * N2 + rbase, RPT)])

    return k


def _leaky(t):
    return jnp.where(t > 0, t, 0.2 * t)


def _build_tc(interpret=False):
    tc = {}

    def a1_body(x_ref, parts_ref, w1_ref, as_ref, ad_ref, wv1_ref, wv2_ref,
                t1_ref, adt_ref, aself_ref, ael2_ref):
        h1 = jnp.dot(x_ref[...], w1_ref[...], preferred_element_type=jnp.float32)
        asrc = h1 @ as_ref[...]
        adst = h1 @ ad_ref[...]
        agg = parts_ref[0:N, :] + parts_ref[N2:N2 + N, :]
        la = agg[:, 0:16] / jnp.maximum(agg[:, 16:17], 1.0)
        ael1 = la @ wv1_ref[...]
        ael2 = la @ wv2_ref[...]
        aself_ref[...] = jnp.exp(_leaky(asrc + adst + ael1))
        ael2_ref[...] = ael2
        adt_ref[...] = adst
        t1_ref[...] = jnp.concatenate([h1, asrc], axis=1)

    tc['a1'] = pl.pallas_call(
        a1_body,
        out_shape=(jax.ShapeDtypeStruct((N, 17), jnp.float32),
                   jax.ShapeDtypeStruct((N, 1), jnp.float32),
                   jax.ShapeDtypeStruct((N, 1), jnp.float32),
                   jax.ShapeDtypeStruct((N, 1), jnp.float32)),
        interpret=interpret)

    def a2_body(ea_ref, wv12_ref, out_ref):
        out_ref[...] = jnp.dot(ea_ref[...], wv12_ref[...],
                               preferred_element_type=jnp.float32)

    tc['a2'] = pl.pallas_call(
        a2_body,
        grid=(NBLK,),
        in_specs=[pl.BlockSpec((BLK, 16), lambda i: (i, 0)),
                  pl.BlockSpec((16, 2), lambda i: (0, 0))],
        out_specs=pl.BlockSpec((BLK, 2), lambda i: (i, 0)),
        out_shape=jax.ShapeDtypeStruct((EP, 2), jnp.float32),
        interpret=interpret)

    def _make_b(F, col):
        def b_body(hs_ref, adt_ref, ae_ref, c_ref):
            hs = hs_ref[...]
            t = hs[:, F:F + 1] + adt_ref[...] + ae_ref[:, col:col + 1]
            p = jnp.exp(_leaky(t))
            c_ref[...] = jnp.concatenate([p * hs[:, 0:F], p], axis=1)

        return pl.pallas_call(
            b_body,
            grid=(NBLK,),
            in_specs=[pl.BlockSpec((BLK, F + 1), lambda i: (i, 0)),
                      pl.BlockSpec((BLK, 1), lambda i: (i, 0)),
                      pl.BlockSpec((BLK, 2), lambda i: (i, 0))],
            out_specs=pl.BlockSpec((BLK, F + 1), lambda i: (i, 0)),
            out_shape=jax.ShapeDtypeStruct((EP, F + 1), jnp.float32),
            interpret=interpret)

    tc['b16'] = _make_b(16, 0)
    tc['b32'] = _make_b(32, 1)

    def c_body(parts_ref, t1_ref, aself_ref, ael2_ref, b1_ref, g_ref, bb_ref,
               w2_ref, as2_ref, ad2_ref, t2_ref, adt2_ref, aself2_ref):
        agg = parts_ref[0:N, :] + parts_ref[N2:N2 + N, :]
        h1 = t1_ref[:, 0:16]
        aself = aself_ref[...]
        out1 = (agg[:, 0:16] + aself * h1) / (agg[:, 16:17] + aself + 1e-16)
        out1 = out1 + b1_ref[...]
        mu = jnp.mean(out1, axis=0, keepdims=True)
        var = jnp.mean((out1 - mu) ** 2, axis=0, keepdims=True)
        h1b = (out1 - mu) / jnp.sqrt(var + 1e-5) * g_ref[...] + bb_ref[...]
        h2 = jnp.dot(h1b, w2_ref[...], preferred_element_type=jnp.float32)
        asrc2 = h2 @ as2_ref[...]
        adst2 = h2 @ ad2_ref[...]
        aself2_ref[...] = jnp.exp(_leaky(asrc2 + adst2 + ael2_ref[...]))
        adt2_ref[...] = adst2
        t2_ref[...] = jnp.concatenate([h2, asrc2], axis=1)

    tc['c'] = pl.pallas_call(
        c_body,
        out_shape=(jax.ShapeDtypeStruct((N, 33), jnp.float32),
                   jax.ShapeDtypeStruct((N, 1), jnp.float32),
                   jax.ShapeDtypeStruct((N, 1), jnp.float32)),
        interpret=interpret)

    def e_body(parts_ref, t2_ref, aself2_ref, b2_ref, hout_ref):
        agg = parts_ref[0:N, :] + parts_ref[N2:N2 + N, :]
        h2 = t2_ref[:, 0:32]
        aself = aself2_ref[...]
        hout = (agg[:, 0:32] + aself * h2) / (agg[:, 32:33] + aself + 1e-16)
        hout_ref[...] = hout + b2_ref[...]

    tc['e'] = pl.pallas_call(
        e_body,
        out_shape=jax.ShapeDtypeStruct((N, 32), jnp.float32),
        interpret=interpret)

    def f_body(gs_ref, gd_ref, ea_ref, wms_ref, wmd_ref, wme_ref, bm1_ref,
               wm2_ref, bm2_ref, wdy_ref, wde_ref, bd12_ref,
               z_ref, st_ref):
        i = pl.program_id(0)
        ea = ea_ref[...]
        t = (jnp.dot(gs_ref[...], wms_ref[...], preferred_element_type=jnp.float32)
             + jnp.dot(gd_ref[...], wmd_ref[...], preferred_element_type=jnp.float32)
             + jnp.dot(ea, wme_ref[...], preferred_element_type=jnp.float32)
             + bm1_ref[...])
        y = jnp.dot(jnp.maximum(t, 0.0), wm2_ref[...],
                    preferred_element_type=jnp.float32) + bm2_ref[...]
        z = (jnp.dot(y, wdy_ref[...], preferred_element_type=jnp.float32)
             + jnp.dot(ea, wde_ref[...], preferred_element_type=jnp.float32)
             + bd12_ref[...])
        z_ref[...] = z
        rows = lax.broadcasted_iota(jnp.int32, (BLK, 1), 0) + i * BLK
        zm = jnp.where(rows < E, z, 0.0)
        st = jnp.concatenate([jnp.sum(zm, axis=0, keepdims=True),
                              jnp.sum(zm * zm, axis=0, keepdims=True)], axis=0)

        @pl.when(i == 0)
        def _():
            st_ref[...] = st

        @pl.when(i > 0)
        def _():
            st_ref[...] = st_ref[...] + st

    tc['f'] = pl.pallas_call(
        f_body,
        grid=(NBLK,),
        in_specs=[pl.BlockSpec((BLK, 32), lambda i: (i, 0)),
                  pl.BlockSpec((BLK, 32), lambda i: (i, 0)),
                  pl.BlockSpec((BLK, 16), lambda i: (i, 0)),
                  pl.BlockSpec((32, 32), lambda i: (0, 0)),
                  pl.BlockSpec((32, 32), lambda i: (0, 0)),
                  pl.BlockSpec((16, 32), lambda i: (0, 0)),
                  pl.BlockSpec((1, 32), lambda i: (0, 0)),
                  pl.BlockSpec((32, 32), lambda i: (0, 0)),
                  pl.BlockSpec((1, 32), lambda i: (0, 0)),
                  pl.BlockSpec((32, 32), lambda i: (0, 0)),
                  pl.BlockSpec((16, 32), lambda i: (0, 0)),
                  pl.BlockSpec((1, 32), lambda i: (0, 0))],
        out_specs=(pl.BlockSpec((BLK, 32), lambda i: (i, 0)),
                   pl.BlockSpec((2, 32), lambda i: (0, 0))),
        out_shape=(jax.ShapeDtypeStruct((EP, 32), jnp.float32),
                   jax.ShapeDtypeStruct((2, 32), jnp.float32)),
        interpret=interpret)

    def g_body(z_ref, zst_ref, g_ref, b_ref, wd34_ref, bd34_ref, u_ref, st_ref):
        i = pl.program_id(0)
        mu = zst_ref[0:1, :] / E
        var = zst_ref[1:2, :] / E - mu * mu
        s1 = g_ref[...] / jnp.sqrt(var + 1e-5)
        t1 = b_ref[...] - mu * s1
        zn = jnp.maximum(z_ref[...] * s1 + t1, 0.0)
        u = jnp.dot(zn, wd34_ref[...], preferred_element_type=jnp.float32) + bd34_ref[...]
        u_ref[...] = u
        rows = lax.broadcasted_iota(jnp.int32, (BLK, 1), 0) + i * BLK
        um = jnp.where(rows < E, u, 0.0)
        st = jnp.concatenate([jnp.sum(um, axis=0, keepdims=True),
                              jnp.sum(um * um, axis=0, keepdims=True)], axis=0)

        @pl.when(i == 0)
        def _():
            st_ref[...] = st

        @pl.when(i > 0)
        def _():
            st_ref[...] = st_ref[...] + st

    tc['g'] = pl.pallas_call(
        g_body,
        grid=(NBLK,),
        in_specs=[pl.BlockSpec((BLK, 32), lambda i: (i, 0)),
                  pl.BlockSpec((2, 32), lambda i: (0, 0)),
                  pl.BlockSpec((1, 32), lambda i: (0, 0)),
                  pl.BlockSpec((1, 32), lambda i: (0, 0)),
                  pl.BlockSpec((32, 16), lambda i: (0, 0)),
                  pl.BlockSpec((1, 16), lambda i: (0, 0))],
        out_specs=(pl.BlockSpec((BLK, 16), lambda i: (i, 0)),
                   pl.BlockSpec((2, 16), lambda i: (0, 0))),
        out_shape=(jax.ShapeDtypeStruct((EP, 16), jnp.float32),
                   jax.ShapeDtypeStruct((2, 16), jnp.float32)),
        interpret=interpret)

    def h_body(u_ref, ust_ref, g_ref, b_ref, wd56_ref, bd56_ref, out_ref):
        mu = ust_ref[0:1, :] / E
        var = ust_ref[1:2, :] / E - mu * mu
        s2 = g_ref[...] / jnp.sqrt(var + 1e-5)
        t2 = b_ref[...] - mu * s2
        un = jnp.maximum(u_ref[...] * s2 + t2, 0.0)
        v = jnp.dot(un, wd56_ref[...], preferred_element_type=jnp.float32) + bd56_ref[...]
        out_ref[...] = 1.0 / (1.0 + jnp.exp(-v))

    tc['h'] = pl.pallas_call(
        h_body,
        grid=(NBLK,),
        in_specs=[pl.BlockSpec((BLK, 16), lambda i: (i, 0)),
                  pl.BlockSpec((2, 16), lambda i: (0, 0)),
                  pl.BlockSpec((1, 16), lambda i: (0, 0)),
                  pl.BlockSpec((1, 16), lambda i: (0, 0)),
                  pl.BlockSpec((16, 1), lambda i: (0, 0)),
                  pl.BlockSpec((1, 1), lambda i: (0, 0))],
        out_specs=pl.BlockSpec((BLK, 1), lambda i: (i, 0)),
        out_shape=jax.ShapeDtypeStruct((EP, 1), jnp.float32),
        interpret=interpret)

    return tc


_TC = _build_tc()
_SC = {
    'g17': _make_gather(17, 1),
    'g33': _make_gather(33, 1),
    'g32': _make_gather(32, 32),
    's17': _make_scatter(17),
    's33': _make_scatter(33),
}


def _pipeline(x, edge_index, edge_attr, params, tc, sc):
    p = params
    src = edge_index[0]
    dst = edge_index[1]
    pad_e = EP - E
    src_p = jnp.pad(src, (0, pad_e)).astype(jnp.int32)
    dst_p = jnp.pad(dst, (0, pad_e)).astype(jnp.int32)
    ea_p = jnp.pad(edge_attr, ((0, pad_e), (0, 0)))
    ea1_p = jnp.pad(jnp.concatenate(
        [edge_attr, jnp.ones((E, 1), jnp.float32)], axis=1),
        ((0, pad_e), (0, 0)))
    z17 = jnp.zeros((N2, 17), jnp.float32)
    z33 = jnp.zeros((N2, 33), jnp.float32)

    wv1 = (p['We1'] @ p['att_edge1']).reshape(16, 1)
    wv2 = (p['We2'] @ p['att_edge2']).reshape(16, 1)
    wv12 = jnp.concatenate([wv1, wv2], axis=1)
    wd12 = p['Wd1'] @ p['Wd2']
    bd12 = (p['bd1'] @ p['Wd2'] + p['bd2']).reshape(1, 32)
    wd34 = p['Wd3'] @ p['Wd4']
    bd34 = (p['bd3'] @ p['Wd4'] + p['bd4']).reshape(1, 16)
    wd56 = p['Wd5'] @ p['Wd6']
    bd56 = (p['bd5'] @ p['Wd6'] + p['bd6']).reshape(1, 1)

    parts0 = sc['s17'](ea1_p, dst_p, z17)

    t1, adt1, aself1, ael2 = tc['a1'](
        x, parts0, p['W1'], p['att_src1'].reshape(16, 1),
        p['att_dst1'].reshape(16, 1), wv1, wv2)
    ae12 = tc['a2'](ea_p, wv12)

    hs1, adt1g = sc['g17'](t1, adt1, src_p, dst_p)
    c1 = tc['b16'](hs1, adt1g, ae12)
    parts1 = sc['s17'](c1, dst_p, z17)

    t2, adt2, aself2 = tc['c'](
        parts1, t1, aself1, ael2, p['b1'].reshape(1, 16),
        p['bn16_g'].reshape(1, 16), p['bn16_b'].reshape(1, 16),
        p['W2'], p['att_src2'].reshape(32, 1), p['att_dst2'].reshape(32, 1))

    hs2, adt2g = sc['g33'](t2, adt2, src_p, dst_p)
    c2 = tc['b32'](hs2, adt2g, ae12)
    parts2 = sc['s33'](c2, dst_p, z33)
    hout = tc['e'](parts2, t2, aself2, p['b2'].reshape(1, 32))

    gs, gd = sc['g32'](hout, hout, src_p, dst_p)
    z, zst = tc['f'](gs, gd, ea_p,
                     p['Wm1'][0:32], p['Wm1'][32:64], p['Wm1'][64:80],
                     p['bm1'].reshape(1, 32), p['Wm2'], p['bm2'].reshape(1, 32),
                     wd12[0:32], wd12[32:48], bd12)
    u, ust = tc['g'](z, zst, p['bnd32_g'].reshape(1, 32),
                     p['bnd32_b'].reshape(1, 32), wd34, bd34)
    outp = tc['h'](u, ust, p['bnd16_g'].reshape(1, 16),
                   p['bnd16_b'].reshape(1, 16), wd56, bd56)
    return outp[:E]


def kernel(x, edge_index, edge_attr, params, P, D, K):
    return _pipeline(x, edge_index, edge_attr, params, _TC, _SC)


# speedup vs baseline: 5.3650x; 5.3650x over previous
import functools

import jax
import jax.numpy as jnp
from jax import lax
from jax.experimental import pallas as pl
from jax.experimental.pallas import tpu as pltpu
from jax.experimental.pallas import tpu_sc as plsc

N = 10000
E = 320000
NC, NS = 2, 16
NW = NC * NS
CHUNK = 128
CPW = 80
EP = NW * CHUNK * CPW
N2 = 10240
RPT = N2 // NS
BLK = 2048
NBLK = EP // BLK


def _sc_mesh():
    return plsc.VectorSubcoreMesh(core_axis_name="c", subcore_axis_name="s",
                                  num_cores=NC, num_subcores=NS)


def _make_gather(WA, WB):
    @functools.partial(
        pl.kernel,
        out_type=(jax.ShapeDtypeStruct((EP, WA), jnp.float32),
                  jax.ShapeDtypeStruct((EP, WB), jnp.float32)),
        mesh=_sc_mesh(),
        compiler_params=pltpu.CompilerParams(use_tc_tiling_on_sc=False),
        scratch_types=[
            pltpu.VMEM((CHUNK,), jnp.int32),
            pltpu.VMEM((CHUNK,), jnp.int32),
            pltpu.VMEM((CHUNK, WA), jnp.float32),
            pltpu.VMEM((CHUNK, WB), jnp.float32),
            pltpu.SemaphoreType.DMA,
            pltpu.SemaphoreType.DMA,
        ],
    )
    def k(ta, tb, ia_hbm, ib_hbm, oa, ob, ia_v, ib_v, ra_v, rb_v, sa, sb):
        cid = lax.axis_index("c")
        sid = lax.axis_index("s")
        wid = sid * NC + cid
        base_w = wid * (CPW * CHUNK)

        def body(i, carry):
            base = base_w + i * CHUNK
            pltpu.sync_copy(ia_hbm.at[pl.ds(base, CHUNK)], ia_v)
            pltpu.sync_copy(ib_hbm.at[pl.ds(base, CHUNK)], ib_v)
            ca = pltpu.async_copy(ta.at[ia_v], ra_v, sa)
            cb = pltpu.async_copy(tb.at[ib_v], rb_v, sb)
            ca.wait()
            cb.wait()
            pltpu.sync_copy(ra_v, oa.at[pl.ds(base, CHUNK)])
            pltpu.sync_copy(rb_v, ob.at[pl.ds(base, CHUNK)])
            return carry

        lax.fori_loop(0, CPW, body, 0)

    return k


def _make_scatter(W):
    @functools.partial(
        pl.kernel,
        out_type=jax.ShapeDtypeStruct((NC * N2, W), jnp.float32),
        mesh=_sc_mesh(),
        compiler_params=pltpu.CompilerParams(use_tc_tiling_on_sc=False),
        scratch_types=[
            pltpu.VMEM((CHUNK,), jnp.int32),
            pltpu.VMEM((CHUNK, W), jnp.float32),
            pltpu.VMEM_SHARED((N2, W), jnp.float32),
            pltpu.SemaphoreType.DMA,
        ],
    )
    def k(pay_hbm, dst_hbm, zeros_hbm, out, idx_v, pay_v, accum, sem):
        cid = lax.axis_index("c")
        sid = lax.axis_index("s")
        wid = sid * NC + cid
        rbase = sid * RPT
        pltpu.sync_copy(zeros_hbm.at[pl.ds(rbase, RPT)],
                        accum.at[pl.ds(rbase, RPT)])
        plsc.subcore_barrier()
        base_w = wid * (CPW * CHUNK)
        nreal = jnp.minimum(CPW, (E - base_w) // CHUNK)

        def body(i, carry):
            base = base_w + i * CHUNK
            pltpu.sync_copy(dst_hbm.at[pl.ds(base, CHUNK)], idx_v)
            pltpu.sync_copy(pay_hbm.at[pl.ds(base, CHUNK)], pay_v)
            pltpu.sync_copy(pay_v, accum.at[idx_v], add=True)
            return carry

        lax.fori_loop(0, nreal, body, 0)
        plsc.subcore_barrier()
        pltpu.sync_copy(accum.at[pl.ds(rbase, RPT)],
                        out.at[pl.ds(cid * N2 + rbase, RPT)])

    return k


def _leaky(t):
    return jnp.where(t > 0, t, 0.2 * t)


def _build_tc(interpret=False):
    tc = {}

    def a1_body(x_ref, parts_ref, w1_ref, as_ref, ad_ref, wv1_ref, wv2_ref,
                t1_ref, adt_ref, aself_ref, ael2_ref):
        h1 = jnp.dot(x_ref[...], w1_ref[...], preferred_element_type=jnp.float32)
        asrc = h1 @ as_ref[...]
        adst = h1 @ ad_ref[...]
        agg = parts_ref[0:N, :] + parts_ref[N2:N2 + N, :]
        la = agg[:, 0:16] / jnp.maximum(agg[:, 16:17], 1.0)
        ael1 = la @ wv1_ref[...]
        ael2 = la @ wv2_ref[...]
        aself_ref[...] = jnp.exp(_leaky(asrc + adst + ael1))
        ael2_ref[...] = ael2
        adt_ref[...] = adst
        t1_ref[...] = jnp.concatenate([h1, asrc], axis=1)

    tc['a1'] = pl.pallas_call(
        a1_body,
        out_shape=(jax.ShapeDtypeStruct((N, 17), jnp.float32),
                   jax.ShapeDtypeStruct((N, 1), jnp.float32),
                   jax.ShapeDtypeStruct((N, 1), jnp.float32),
                   jax.ShapeDtypeStruct((N, 1), jnp.float32)),
        interpret=interpret)

    def a2_body(ea_ref, wv12_ref, out_ref):
        out_ref[...] = jnp.dot(ea_ref[...], wv12_ref[...],
                               preferred_element_type=jnp.float32)

    tc['a2'] = pl.pallas_call(
        a2_body,
        grid=(NBLK,),
        in_specs=[pl.BlockSpec((BLK, 16), lambda i: (i, 0)),
                  pl.BlockSpec((16, 2), lambda i: (0, 0))],
        out_specs=pl.BlockSpec((BLK, 2), lambda i: (i, 0)),
        out_shape=jax.ShapeDtypeStruct((EP, 2), jnp.float32),
        interpret=interpret)

    def _make_b(F, col):
        def b_body(hs_ref, adt_ref, ae_ref, c_ref):
            hs = hs_ref[...]
            t = hs[:, F:F + 1] + adt_ref[...] + ae_ref[:, col:col + 1]
            p = jnp.exp(_leaky(t))
            c_ref[...] = jnp.concatenate([p * hs[:, 0:F], p], axis=1)

        return pl.pallas_call(
            b_body,
            grid=(NBLK,),
            in_specs=[pl.BlockSpec((BLK, F + 1), lambda i: (i, 0)),
                      pl.BlockSpec((BLK, 1), lambda i: (i, 0)),
                      pl.BlockSpec((BLK, 2), lambda i: (i, 0))],
            out_specs=pl.BlockSpec((BLK, F + 1), lambda i: (i, 0)),
            out_shape=jax.ShapeDtypeStruct((EP, F + 1), jnp.float32),
            interpret=interpret)

    tc['b16'] = _make_b(16, 0)
    tc['b32'] = _make_b(32, 1)

    def c_body(parts_ref, t1_ref, aself_ref, ael2_ref, b1_ref, g_ref, bb_ref,
               w2_ref, as2_ref, ad2_ref, t2_ref, adt2_ref, aself2_ref):
        agg = parts_ref[0:N, :] + parts_ref[N2:N2 + N, :]
        h1 = t1_ref[:, 0:16]
        aself = aself_ref[...]
        out1 = (agg[:, 0:16] + aself * h1) / (agg[:, 16:17] + aself + 1e-16)
        out1 = out1 + b1_ref[...]
        mu = jnp.mean(out1, axis=0, keepdims=True)
        var = jnp.mean((out1 - mu) ** 2, axis=0, keepdims=True)
        h1b = (out1 - mu) / jnp.sqrt(var + 1e-5) * g_ref[...] + bb_ref[...]
        h2 = jnp.dot(h1b, w2_ref[...], preferred_element_type=jnp.float32)
        asrc2 = h2 @ as2_ref[...]
        adst2 = h2 @ ad2_ref[...]
        aself2_ref[...] = jnp.exp(_leaky(asrc2 + adst2 + ael2_ref[...]))
        adt2_ref[...] = adst2
        t2_ref[...] = jnp.concatenate([h2, asrc2], axis=1)

    tc['c'] = pl.pallas_call(
        c_body,
        out_shape=(jax.ShapeDtypeStruct((N, 33), jnp.float32),
                   jax.ShapeDtypeStruct((N, 1), jnp.float32),
                   jax.ShapeDtypeStruct((N, 1), jnp.float32)),
        interpret=interpret)

    def e_body(parts_ref, t2_ref, aself2_ref, b2_ref, hout_ref):
        agg = parts_ref[0:N, :] + parts_ref[N2:N2 + N, :]
        h2 = t2_ref[:, 0:32]
        aself = aself2_ref[...]
        hout = (agg[:, 0:32] + aself * h2) / (agg[:, 32:33] + aself + 1e-16)
        hout_ref[...] = hout + b2_ref[...]

    tc['e'] = pl.pallas_call(
        e_body,
        out_shape=jax.ShapeDtypeStruct((N, 32), jnp.float32),
        interpret=interpret)

    def f_body(gs_ref, gd_ref, ea_ref, wms_ref, wmd_ref, wme_ref, bm1_ref,
               wm2_ref, bm2_ref, wdy_ref, wde_ref, bd12_ref,
               z_ref, st_ref):
        i = pl.program_id(0)
        ea = ea_ref[...]
        t = (jnp.dot(gs_ref[...], wms_ref[...], preferred_element_type=jnp.float32)
             + jnp.dot(gd_ref[...], wmd_ref[...], preferred_element_type=jnp.float32)
             + jnp.dot(ea, wme_ref[...], preferred_element_type=jnp.float32)
             + bm1_ref[...])
        y = jnp.dot(jnp.maximum(t, 0.0), wm2_ref[...],
                    preferred_element_type=jnp.float32) + bm2_ref[...]
        z = (jnp.dot(y, wdy_ref[...], preferred_element_type=jnp.float32)
             + jnp.dot(ea, wde_ref[...], preferred_element_type=jnp.float32)
             + bd12_ref[...])
        z_ref[...] = z
        rows = lax.broadcasted_iota(jnp.int32, (BLK, 1), 0) + i * BLK
        zm = jnp.where(rows < E, z, 0.0)
        st = jnp.concatenate([jnp.sum(zm, axis=0, keepdims=True),
                              jnp.sum(zm * zm, axis=0, keepdims=True)], axis=0)

        @pl.when(i == 0)
        def _():
            st_ref[...] = st

        @pl.when(i > 0)
        def _():
            st_ref[...] = st_ref[...] + st

    tc['f'] = pl.pallas_call(
        f_body,
        grid=(NBLK,),
        in_specs=[pl.BlockSpec((BLK, 32), lambda i: (i, 0)),
                  pl.BlockSpec((BLK, 32), lambda i: (i, 0)),
                  pl.BlockSpec((BLK, 16), lambda i: (i, 0)),
                  pl.BlockSpec((32, 32), lambda i: (0, 0)),
                  pl.BlockSpec((32, 32), lambda i: (0, 0)),
                  pl.BlockSpec((16, 32), lambda i: (0, 0)),
                  pl.BlockSpec((1, 32), lambda i: (0, 0)),
                  pl.BlockSpec((32, 32), lambda i: (0, 0)),
                  pl.BlockSpec((1, 32), lambda i: (0, 0)),
                  pl.BlockSpec((32, 32), lambda i: (0, 0)),
                  pl.BlockSpec((16, 32), lambda i: (0, 0)),
                  pl.BlockSpec((1, 32), lambda i: (0, 0))],
        out_specs=(pl.BlockSpec((BLK, 32), lambda i: (i, 0)),
                   pl.BlockSpec((2, 32), lambda i: (0, 0))),
        out_shape=(jax.ShapeDtypeStruct((EP, 32), jnp.float32),
                   jax.ShapeDtypeStruct((2, 32), jnp.float32)),
        interpret=interpret)

    def g_body(z_ref, zst_ref, g_ref, b_ref, wd34_ref, bd34_ref, u_ref, st_ref):
        i = pl.program_id(0)
        mu = zst_ref[0:1, :] / E
        var = zst_ref[1:2, :] / E - mu * mu
        s1 = g_ref[...] / jnp.sqrt(var + 1e-5)
        t1 = b_ref[...] - mu * s1
        zn = jnp.maximum(z_ref[...] * s1 + t1, 0.0)
        u = jnp.dot(zn, wd34_ref[...], preferred_element_type=jnp.float32) + bd34_ref[...]
        u_ref[...] = u
        rows = lax.broadcasted_iota(jnp.int32, (BLK, 1), 0) + i * BLK
        um = jnp.where(rows < E, u, 0.0)
        st = jnp.concatenate([jnp.sum(um, axis=0, keepdims=True),
                              jnp.sum(um * um, axis=0, keepdims=True)], axis=0)

        @pl.when(i == 0)
        def _():
            st_ref[...] = st

        @pl.when(i > 0)
        def _():
            st_ref[...] = st_ref[...] + st

    tc['g'] = pl.pallas_call(
        g_body,
        grid=(NBLK,),
        in_specs=[pl.BlockSpec((BLK, 32), lambda i: (i, 0)),
                  pl.BlockSpec((2, 32), lambda i: (0, 0)),
                  pl.BlockSpec((1, 32), lambda i: (0, 0)),
                  pl.BlockSpec((1, 32), lambda i: (0, 0)),
                  pl.BlockSpec((32, 16), lambda i: (0, 0)),
                  pl.BlockSpec((1, 16), lambda i: (0, 0))],
        out_specs=(pl.BlockSpec((BLK, 16), lambda i: (i, 0)),
                   pl.BlockSpec((2, 16), lambda i: (0, 0))),
        out_shape=(jax.ShapeDtypeStruct((EP, 16), jnp.float32),
                   jax.ShapeDtypeStruct((2, 16), jnp.float32)),
        interpret=interpret)

    def h_body(u_ref, ust_ref, g_ref, b_ref, wd56_ref, bd56_ref, out_ref):
        mu = ust_ref[0:1, :] / E
        var = ust_ref[1:2, :] / E - mu * mu
        s2 = g_ref[...] / jnp.sqrt(var + 1e-5)
        t2 = b_ref[...] - mu * s2
        un = jnp.maximum(u_ref[...] * s2 + t2, 0.0)
        v = jnp.dot(un, wd56_ref[...], preferred_element_type=jnp.float32) + bd56_ref[...]
        out_ref[...] = 1.0 / (1.0 + jnp.exp(-v))

    tc['h'] = pl.pallas_call(
        h_body,
        grid=(NBLK,),
        in_specs=[pl.BlockSpec((BLK, 16), lambda i: (i, 0)),
                  pl.BlockSpec((2, 16), lambda i: (0, 0)),
                  pl.BlockSpec((1, 16), lambda i: (0, 0)),
                  pl.BlockSpec((1, 16), lambda i: (0, 0)),
                  pl.BlockSpec((16, 1), lambda i: (0, 0)),
                  pl.BlockSpec((1, 1), lambda i: (0, 0))],
        out_specs=pl.BlockSpec((BLK, 1), lambda i: (i, 0)),
        out_shape=jax.ShapeDtypeStruct((EP, 1), jnp.float32),
        interpret=interpret)

    return tc


_IMPL = []


def _get_impl():
    if not _IMPL:
        tc = _build_tc()
        sc = {
            'g17': _make_gather(17, 1),
            'g33': _make_gather(33, 1),
            'g32': _make_gather(32, 32),
            's17': _make_scatter(17),
            's33': _make_scatter(33),
        }
        _IMPL.append((tc, sc))
    return _IMPL[0]


def _pipeline(x, edge_index, edge_attr, params, tc, sc):
    p = params
    src = edge_index[0]
    dst = edge_index[1]
    pad_e = EP - E
    src_p = jnp.pad(src, (0, pad_e)).astype(jnp.int32)
    dst_p = jnp.pad(dst, (0, pad_e)).astype(jnp.int32)
    ea_p = jnp.pad(edge_attr, ((0, pad_e), (0, 0)))
    ea1_p = jnp.pad(jnp.concatenate(
        [edge_attr, jnp.ones((E, 1), jnp.float32)], axis=1),
        ((0, pad_e), (0, 0)))
    z17 = jnp.zeros((N2, 17), jnp.float32)
    z33 = jnp.zeros((N2, 33), jnp.float32)

    wv1 = (p['We1'] @ p['att_edge1']).reshape(16, 1)
    wv2 = (p['We2'] @ p['att_edge2']).reshape(16, 1)
    wv12 = jnp.concatenate([wv1, wv2], axis=1)
    wd12 = p['Wd1'] @ p['Wd2']
    bd12 = (p['bd1'] @ p['Wd2'] + p['bd2']).reshape(1, 32)
    wd34 = p['Wd3'] @ p['Wd4']
    bd34 = (p['bd3'] @ p['Wd4'] + p['bd4']).reshape(1, 16)
    wd56 = p['Wd5'] @ p['Wd6']
    bd56 = (p['bd5'] @ p['Wd6'] + p['bd6']).reshape(1, 1)

    parts0 = sc['s17'](ea1_p, dst_p, z17)

    t1, adt1, aself1, ael2 = tc['a1'](
        x, parts0, p['W1'], p['att_src1'].reshape(16, 1),
        p['att_dst1'].reshape(16, 1), wv1, wv2)
    ae12 = tc['a2'](ea_p, wv12)

    hs1, adt1g = sc['g17'](t1, adt1, src_p, dst_p)
    c1 = tc['b16'](hs1, adt1g, ae12)
    parts1 = sc['s17'](c1, dst_p, z17)

    t2, adt2, aself2 = tc['c'](
        parts1, t1, aself1, ael2, p['b1'].reshape(1, 16),
        p['bn16_g'].reshape(1, 16), p['bn16_b'].reshape(1, 16),
        p['W2'], p['att_src2'].reshape(32, 1), p['att_dst2'].reshape(32, 1))

    hs2, adt2g = sc['g33'](t2, adt2, src_p, dst_p)
    c2 = tc['b32'](hs2, adt2g, ae12)
    parts2 = sc['s33'](c2, dst_p, z33)
    hout = tc['e'](parts2, t2, aself2, p['b2'].reshape(1, 32))

    gs, gd = sc['g32'](hout, hout, src_p, dst_p)
    z, zst = tc['f'](gs, gd, ea_p,
                     p['Wm1'][0:32], p['Wm1'][32:64], p['Wm1'][64:80],
                     p['bm1'].reshape(1, 32), p['Wm2'], p['bm2'].reshape(1, 32),
                     wd12[0:32], wd12[32:48], bd12)
    u, ust = tc['g'](z, zst, p['bnd32_g'].reshape(1, 32),
                     p['bnd32_b'].reshape(1, 32), wd34, bd34)
    outp = tc['h'](u, ust, p['bnd16_g'].reshape(1, 16),
                   p['bnd16_b'].reshape(1, 16), wd56, bd56)
    return outp[:E]


def kernel(x, edge_index, edge_attr, params, P, D, K):
    tc, sc = _get_impl()
    return _pipeline(x, edge_index, edge_attr, params, tc, sc)


# speedup vs baseline: 5.9325x; 1.1058x over previous
import functools

import jax
import jax.numpy as jnp
from jax import lax
from jax.experimental import pallas as pl
from jax.experimental.pallas import tpu as pltpu
from jax.experimental.pallas import tpu_sc as plsc

N = 10000
E = 320000
NC, NS = 2, 16
NW = NC * NS
CHUNK = 128
CPW = 80
EP = NW * CHUNK * CPW
N2 = 10240
RPT = N2 // NS
BLK = 2048
NBLK = EP // BLK


def _sc_mesh():
    return plsc.VectorSubcoreMesh(core_axis_name="c", subcore_axis_name="s",
                                  num_cores=NC, num_subcores=NS)


SUP = 1280
KC = SUP // CHUNK
NSUP = CPW * CHUNK // SUP


def _make_gather(WA, WB):
    @functools.partial(
        pl.kernel,
        out_type=(jax.ShapeDtypeStruct((EP, WA), jnp.float32),
                  jax.ShapeDtypeStruct((EP, WB), jnp.float32)),
        mesh=_sc_mesh(),
        compiler_params=pltpu.CompilerParams(use_tc_tiling_on_sc=False),
        scratch_types=[
            pltpu.VMEM((CPW, CHUNK), jnp.int32),
            pltpu.VMEM((CPW, CHUNK), jnp.int32),
            pltpu.VMEM((SUP, WA), jnp.float32),
            pltpu.VMEM((SUP, WB), jnp.float32),
            pltpu.SemaphoreType.DMA,
            pltpu.SemaphoreType.DMA,
            pltpu.SemaphoreType.DMA,
        ],
    )
    def k(ta, tb, ia_hbm, ib_hbm, oa, ob, ia_v, ib_v, ra_v, rb_v, sa, sb, sw):
        cid = lax.axis_index("c")
        sid = lax.axis_index("s")
        wid = sid * NC + cid
        base_w = wid * (CPW * CHUNK)
        crow = wid * CPW
        pltpu.sync_copy(ia_hbm.at[pl.ds(crow, CPW)], ia_v)
        pltpu.sync_copy(ib_hbm.at[pl.ds(crow, CPW)], ib_v)

        def body(s, carry):
            descs = []
            for j in range(KC):
                descs.append(pltpu.async_copy(
                    ta.at[ia_v.at[s * KC + j]],
                    ra_v.at[pl.ds(j * CHUNK, CHUNK)], sa))
                descs.append(pltpu.async_copy(
                    tb.at[ib_v.at[s * KC + j]],
                    rb_v.at[pl.ds(j * CHUNK, CHUNK)], sb))
            for d in descs:
                d.wait()
            base = base_w + s * SUP
            wa = pltpu.async_copy(ra_v, oa.at[pl.ds(base, SUP)], sw)
            wb = pltpu.async_copy(rb_v, ob.at[pl.ds(base, SUP)], sw)
            wa.wait()
            wb.wait()
            return carry

        lax.fori_loop(0, NSUP, body, 0)

    return k


def _make_scatter(W):
    @functools.partial(
        pl.kernel,
        out_type=jax.ShapeDtypeStruct((NC * N2, W), jnp.float32),
        mesh=_sc_mesh(),
        compiler_params=pltpu.CompilerParams(use_tc_tiling_on_sc=False),
        scratch_types=[
            pltpu.VMEM((CPW, CHUNK), jnp.int32),
            pltpu.VMEM((SUP, W), jnp.float32),
            pltpu.VMEM_SHARED((N2, W), jnp.float32),
            pltpu.SemaphoreType.DMA,
            pltpu.SemaphoreType.DMA,
        ],
    )
    def k(pay_hbm, dst_hbm, zeros_hbm, out, idx_v, pay_v, accum, sp, ss):
        cid = lax.axis_index("c")
        sid = lax.axis_index("s")
        wid = sid * NC + cid
        rbase = sid * RPT
        pltpu.sync_copy(zeros_hbm.at[pl.ds(rbase, RPT)],
                        accum.at[pl.ds(rbase, RPT)])
        plsc.subcore_barrier()
        base_w = wid * (CPW * CHUNK)
        crow = wid * CPW
        pltpu.sync_copy(dst_hbm.at[pl.ds(crow, CPW)], idx_v)
        nsup = jnp.minimum(NSUP, (E - base_w) // SUP)

        def body(s, carry):
            base = base_w + s * SUP
            pltpu.sync_copy(pay_hbm.at[pl.ds(base, SUP)], pay_v)
            descs = []
            for j in range(KC):
                descs.append(pltpu.async_copy(
                    pay_v.at[pl.ds(j * CHUNK, CHUNK)],
                    accum.at[idx_v.at[s * KC + j]], ss, add=True))
            for d in descs:
                d.wait()
            return carry

        lax.fori_loop(0, nsup, body, 0)
        plsc.subcore_barrier()
        pltpu.sync_copy(accum.at[pl.ds(rbase, RPT)],
                        out.at[pl.ds(cid * N2 + rbase, RPT)])

    return k


def _leaky(t):
    return jnp.where(t > 0, t, 0.2 * t)


def _build_tc(interpret=False):
    tc = {}

    def a1_body(x_ref, parts_ref, w1_ref, as_ref, ad_ref, wv1_ref, wv2_ref,
                t1_ref, adt_ref, aself_ref, ael2_ref):
        h1 = jnp.dot(x_ref[...], w1_ref[...], preferred_element_type=jnp.float32)
        asrc = h1 @ as_ref[...]
        adst = h1 @ ad_ref[...]
        agg = parts_ref[0:N, :] + parts_ref[N2:N2 + N, :]
        la = agg[:, 0:16] / jnp.maximum(agg[:, 16:17], 1.0)
        ael1 = la @ wv1_ref[...]
        ael2 = la @ wv2_ref[...]
        aself_ref[...] = jnp.exp(_leaky(asrc + adst + ael1))
        ael2_ref[...] = ael2
        adt_ref[...] = adst
        t1_ref[...] = jnp.concatenate([h1, asrc], axis=1)

    tc['a1'] = pl.pallas_call(
        a1_body,
        out_shape=(jax.ShapeDtypeStruct((N, 17), jnp.float32),
                   jax.ShapeDtypeStruct((N, 1), jnp.float32),
                   jax.ShapeDtypeStruct((N, 1), jnp.float32),
                   jax.ShapeDtypeStruct((N, 1), jnp.float32)),
        interpret=interpret)

    def a2_body(ea_ref, wv12_ref, out_ref):
        out_ref[...] = jnp.dot(ea_ref[...], wv12_ref[...],
                               preferred_element_type=jnp.float32)

    tc['a2'] = pl.pallas_call(
        a2_body,
        grid=(NBLK,),
        in_specs=[pl.BlockSpec((BLK, 16), lambda i: (i, 0)),
                  pl.BlockSpec((16, 2), lambda i: (0, 0))],
        out_specs=pl.BlockSpec((BLK, 2), lambda i: (i, 0)),
        out_shape=jax.ShapeDtypeStruct((EP, 2), jnp.float32),
        interpret=interpret)

    def _make_b(F, col):
        def b_body(hs_ref, adt_ref, ae_ref, c_ref):
            hs = hs_ref[...]
            t = hs[:, F:F + 1] + adt_ref[...] + ae_ref[:, col:col + 1]
            p = jnp.exp(_leaky(t))
            c_ref[...] = jnp.concatenate([p * hs[:, 0:F], p], axis=1)

        return pl.pallas_call(
            b_body,
            grid=(NBLK,),
            in_specs=[pl.BlockSpec((BLK, F + 1), lambda i: (i, 0)),
                      pl.BlockSpec((BLK, 1), lambda i: (i, 0)),
                      pl.BlockSpec((BLK, 2), lambda i: (i, 0))],
            out_specs=pl.BlockSpec((BLK, F + 1), lambda i: (i, 0)),
            out_shape=jax.ShapeDtypeStruct((EP, F + 1), jnp.float32),
            interpret=interpret)

    tc['b16'] = _make_b(16, 0)
    tc['b32'] = _make_b(32, 1)

    def c_body(parts_ref, t1_ref, aself_ref, ael2_ref, b1_ref, g_ref, bb_ref,
               w2_ref, as2_ref, ad2_ref, t2_ref, adt2_ref, aself2_ref):
        agg = parts_ref[0:N, :] + parts_ref[N2:N2 + N, :]
        h1 = t1_ref[:, 0:16]
        aself = aself_ref[...]
        out1 = (agg[:, 0:16] + aself * h1) / (agg[:, 16:17] + aself + 1e-16)
        out1 = out1 + b1_ref[...]
        mu = jnp.mean(out1, axis=0, keepdims=True)
        var = jnp.mean((out1 - mu) ** 2, axis=0, keepdims=True)
        h1b = (out1 - mu) / jnp.sqrt(var + 1e-5) * g_ref[...] + bb_ref[...]
        h2 = jnp.dot(h1b, w2_ref[...], preferred_element_type=jnp.float32)
        asrc2 = h2 @ as2_ref[...]
        adst2 = h2 @ ad2_ref[...]
        aself2_ref[...] = jnp.exp(_leaky(asrc2 + adst2 + ael2_ref[...]))
        adt2_ref[...] = adst2
        t2_ref[...] = jnp.concatenate([h2, asrc2], axis=1)

    tc['c'] = pl.pallas_call(
        c_body,
        out_shape=(jax.ShapeDtypeStruct((N, 33), jnp.float32),
                   jax.ShapeDtypeStruct((N, 1), jnp.float32),
                   jax.ShapeDtypeStruct((N, 1), jnp.float32)),
        interpret=interpret)

    def e_body(parts_ref, t2_ref, aself2_ref, b2_ref, hout_ref):
        agg = parts_ref[0:N, :] + parts_ref[N2:N2 + N, :]
        h2 = t2_ref[:, 0:32]
        aself = aself2_ref[...]
        hout = (agg[:, 0:32] + aself * h2) / (agg[:, 32:33] + aself + 1e-16)
        hout_ref[...] = hout + b2_ref[...]

    tc['e'] = pl.pallas_call(
        e_body,
        out_shape=jax.ShapeDtypeStruct((N, 32), jnp.float32),
        interpret=interpret)

    def f_body(gs_ref, gd_ref, ea_ref, wms_ref, wmd_ref, wme_ref, bm1_ref,
               wm2_ref, bm2_ref, wdy_ref, wde_ref, bd12_ref,
               z_ref, st_ref):
        i = pl.program_id(0)
        ea = ea_ref[...]
        t = (jnp.dot(gs_ref[...], wms_ref[...], preferred_element_type=jnp.float32)
             + jnp.dot(gd_ref[...], wmd_ref[...], preferred_element_type=jnp.float32)
             + jnp.dot(ea, wme_ref[...], preferred_element_type=jnp.float32)
             + bm1_ref[...])
        y = jnp.dot(jnp.maximum(t, 0.0), wm2_ref[...],
                    preferred_element_type=jnp.float32) + bm2_ref[...]
        z = (jnp.dot(y, wdy_ref[...], preferred_element_type=jnp.float32)
             + jnp.dot(ea, wde_ref[...], preferred_element_type=jnp.float32)
             + bd12_ref[...])
        z_ref[...] = z
        rows = lax.broadcasted_iota(jnp.int32, (BLK, 1), 0) + i * BLK
        zm = jnp.where(rows < E, z, 0.0)
        st = jnp.concatenate([jnp.sum(zm, axis=0, keepdims=True),
                              jnp.sum(zm * zm, axis=0, keepdims=True)], axis=0)

        @pl.when(i == 0)
        def _():
            st_ref[...] = st

        @pl.when(i > 0)
        def _():
            st_ref[...] = st_ref[...] + st

    tc['f'] = pl.pallas_call(
        f_body,
        grid=(NBLK,),
        in_specs=[pl.BlockSpec((BLK, 32), lambda i: (i, 0)),
                  pl.BlockSpec((BLK, 32), lambda i: (i, 0)),
                  pl.BlockSpec((BLK, 16), lambda i: (i, 0)),
                  pl.BlockSpec((32, 32), lambda i: (0, 0)),
                  pl.BlockSpec((32, 32), lambda i: (0, 0)),
                  pl.BlockSpec((16, 32), lambda i: (0, 0)),
                  pl.BlockSpec((1, 32), lambda i: (0, 0)),
                  pl.BlockSpec((32, 32), lambda i: (0, 0)),
                  pl.BlockSpec((1, 32), lambda i: (0, 0)),
                  pl.BlockSpec((32, 32), lambda i: (0, 0)),
                  pl.BlockSpec((16, 32), lambda i: (0, 0)),
                  pl.BlockSpec((1, 32), lambda i: (0, 0))],
        out_specs=(pl.BlockSpec((BLK, 32), lambda i: (i, 0)),
                   pl.BlockSpec((2, 32), lambda i: (0, 0))),
        out_shape=(jax.ShapeDtypeStruct((EP, 32), jnp.float32),
                   jax.ShapeDtypeStruct((2, 32), jnp.float32)),
        interpret=interpret)

    def g_body(z_ref, zst_ref, g_ref, b_ref, wd34_ref, bd34_ref, u_ref, st_ref):
        i = pl.program_id(0)
        mu = zst_ref[0:1, :] / E
        var = zst_ref[1:2, :] / E - mu * mu
        s1 = g_ref[...] / jnp.sqrt(var + 1e-5)
        t1 = b_ref[...] - mu * s1
        zn = jnp.maximum(z_ref[...] * s1 + t1, 0.0)
        u = jnp.dot(zn, wd34_ref[...], preferred_element_type=jnp.float32) + bd34_ref[...]
        u_ref[...] = u
        rows = lax.broadcasted_iota(jnp.int32, (BLK, 1), 0) + i * BLK
        um = jnp.where(rows < E, u, 0.0)
        st = jnp.concatenate([jnp.sum(um, axis=0, keepdims=True),
                              jnp.sum(um * um, axis=0, keepdims=True)], axis=0)

        @pl.when(i == 0)
        def _():
            st_ref[...] = st

        @pl.when(i > 0)
        def _():
            st_ref[...] = st_ref[...] + st

    tc['g'] = pl.pallas_call(
        g_body,
        grid=(NBLK,),
        in_specs=[pl.BlockSpec((BLK, 32), lambda i: (i, 0)),
                  pl.BlockSpec((2, 32), lambda i: (0, 0)),
                  pl.BlockSpec((1, 32), lambda i: (0, 0)),
                  pl.BlockSpec((1, 32), lambda i: (0, 0)),
                  pl.BlockSpec((32, 16), lambda i: (0, 0)),
                  pl.BlockSpec((1, 16), lambda i: (0, 0))],
        out_specs=(pl.BlockSpec((BLK, 16), lambda i: (i, 0)),
                   pl.BlockSpec((2, 16), lambda i: (0, 0))),
        out_shape=(jax.ShapeDtypeStruct((EP, 16), jnp.float32),
                   jax.ShapeDtypeStruct((2, 16), jnp.float32)),
        interpret=interpret)

    def h_body(u_ref, ust_ref, g_ref, b_ref, wd56_ref, bd56_ref, out_ref):
        mu = ust_ref[0:1, :] / E
        var = ust_ref[1:2, :] / E - mu * mu
        s2 = g_ref[...] / jnp.sqrt(var + 1e-5)
        t2 = b_ref[...] - mu * s2
        un = jnp.maximum(u_ref[...] * s2 + t2, 0.0)
        v = jnp.dot(un, wd56_ref[...], preferred_element_type=jnp.float32) + bd56_ref[...]
        out_ref[...] = 1.0 / (1.0 + jnp.exp(-v))

    tc['h'] = pl.pallas_call(
        h_body,
        grid=(NBLK,),
        in_specs=[pl.BlockSpec((BLK, 16), lambda i: (i, 0)),
                  pl.BlockSpec((2, 16), lambda i: (0, 0)),
                  pl.BlockSpec((1, 16), lambda i: (0, 0)),
                  pl.BlockSpec((1, 16), lambda i: (0, 0)),
                  pl.BlockSpec((16, 1), lambda i: (0, 0)),
                  pl.BlockSpec((1, 1), lambda i: (0, 0))],
        out_specs=pl.BlockSpec((BLK, 1), lambda i: (i, 0)),
        out_shape=jax.ShapeDtypeStruct((EP, 1), jnp.float32),
        interpret=interpret)

    return tc


_IMPL = []


def _get_impl():
    if not _IMPL:
        tc = _build_tc()
        sc = {
            'g17': _make_gather(17, 1),
            'g33': _make_gather(33, 1),
            'g32': _make_gather(32, 32),
            's17': _make_scatter(17),
            's33': _make_scatter(33),
        }
        _IMPL.append((tc, sc))
    return _IMPL[0]


def _pipeline(x, edge_index, edge_attr, params, tc, sc):
    p = params
    src = edge_index[0]
    dst = edge_index[1]
    pad_e = EP - E
    src_p = jnp.pad(src, (0, pad_e)).astype(jnp.int32).reshape(EP // CHUNK, CHUNK)
    dst_p = jnp.pad(dst, (0, pad_e)).astype(jnp.int32).reshape(EP // CHUNK, CHUNK)
    ea_p = jnp.pad(edge_attr, ((0, pad_e), (0, 0)))
    ea1_p = jnp.pad(jnp.concatenate(
        [edge_attr, jnp.ones((E, 1), jnp.float32)], axis=1),
        ((0, pad_e), (0, 0)))
    z17 = jnp.zeros((N2, 17), jnp.float32)
    z33 = jnp.zeros((N2, 33), jnp.float32)

    wv1 = (p['We1'] @ p['att_edge1']).reshape(16, 1)
    wv2 = (p['We2'] @ p['att_edge2']).reshape(16, 1)
    wv12 = jnp.concatenate([wv1, wv2], axis=1)
    wd12 = p['Wd1'] @ p['Wd2']
    bd12 = (p['bd1'] @ p['Wd2'] + p['bd2']).reshape(1, 32)
    wd34 = p['Wd3'] @ p['Wd4']
    bd34 = (p['bd3'] @ p['Wd4'] + p['bd4']).reshape(1, 16)
    wd56 = p['Wd5'] @ p['Wd6']
    bd56 = (p['bd5'] @ p['Wd6'] + p['bd6']).reshape(1, 1)

    parts0 = sc['s17'](ea1_p, dst_p, z17)

    t1, adt1, aself1, ael2 = tc['a1'](
        x, parts0, p['W1'], p['att_src1'].reshape(16, 1),
        p['att_dst1'].reshape(16, 1), wv1, wv2)
    ae12 = tc['a2'](ea_p, wv12)

    hs1, adt1g = sc['g17'](t1, adt1, src_p, dst_p)
    c1 = tc['b16'](hs1, adt1g, ae12)
    parts1 = sc['s17'](c1, dst_p, z17)

    t2, adt2, aself2 = tc['c'](
        parts1, t1, aself1, ael2, p['b1'].reshape(1, 16),
        p['bn16_g'].reshape(1, 16), p['bn16_b'].reshape(1, 16),
        p['W2'], p['att_src2'].reshape(32, 1), p['att_dst2'].reshape(32, 1))

    hs2, adt2g = sc['g33'](t2, adt2, src_p, dst_p)
    c2 = tc['b32'](hs2, adt2g, ae12)
    parts2 = sc['s33'](c2, dst_p, z33)
    hout = tc['e'](parts2, t2, aself2, p['b2'].reshape(1, 32))

    gs, gd = sc['g32'](hout, hout, src_p, dst_p)
    z, zst = tc['f'](gs, gd, ea_p,
                     p['Wm1'][0:32], p['Wm1'][32:64], p['Wm1'][64:80],
                     p['bm1'].reshape(1, 32), p['Wm2'], p['bm2'].reshape(1, 32),
                     wd12[0:32], wd12[32:48], bd12)
    u, ust = tc['g'](z, zst, p['bnd32_g'].reshape(1, 32),
                     p['bnd32_b'].reshape(1, 32), wd34, bd34)
    outp = tc['h'](u, ust, p['bnd16_g'].reshape(1, 16),
                   p['bnd16_b'].reshape(1, 16), wd56, bd56)
    return outp[:E]


def kernel(x, edge_index, edge_attr, params, P, D, K):
    tc, sc = _get_impl()
    return _pipeline(x, edge_index, edge_attr, params, tc, sc)


# speedup vs baseline: 8.3041x; 1.3998x over previous
import functools

import jax
import jax.numpy as jnp
from jax import lax
from jax.experimental import pallas as pl
from jax.experimental.pallas import tpu as pltpu
from jax.experimental.pallas import tpu_sc as plsc

N = 10000
E = 320000
NC, NS = 2, 16
NW = NC * NS
CHUNK = 128
CPW = 80
EP = NW * CHUNK * CPW
N2 = 10240
RPT = N2 // NS
BLK = 2048
NBLK = EP // BLK


def _sc_mesh():
    return plsc.VectorSubcoreMesh(core_axis_name="c", subcore_axis_name="s",
                                  num_cores=NC, num_subcores=NS)


SUP = 1280
KC = SUP // CHUNK
NSUP = CPW * CHUNK // SUP


def _make_gather(WA, WB):
    @functools.partial(
        pl.kernel,
        out_type=(jax.ShapeDtypeStruct((EP, WA), jnp.float32),
                  jax.ShapeDtypeStruct((EP, WB), jnp.float32)),
        mesh=_sc_mesh(),
        compiler_params=pltpu.CompilerParams(use_tc_tiling_on_sc=False),
        scratch_types=[
            pltpu.VMEM((CPW, CHUNK), jnp.int32),
            pltpu.VMEM((CPW, CHUNK), jnp.int32),
            pltpu.VMEM((SUP, WA), jnp.float32),
            pltpu.VMEM((SUP, WB), jnp.float32),
            pltpu.SemaphoreType.DMA,
            pltpu.SemaphoreType.DMA,
            pltpu.SemaphoreType.DMA,
        ],
    )
    def k(ta, tb, ia_hbm, ib_hbm, oa, ob, ia_v, ib_v, ra_v, rb_v, sa, sb, sw):
        cid = lax.axis_index("c")
        sid = lax.axis_index("s")
        wid = sid * NC + cid
        base_w = wid * (CPW * CHUNK)
        crow = wid * CPW
        pltpu.sync_copy(ia_hbm.at[pl.ds(crow, CPW)], ia_v)
        pltpu.sync_copy(ib_hbm.at[pl.ds(crow, CPW)], ib_v)

        def body(s, carry):
            descs = []
            for j in range(KC):
                descs.append(pltpu.async_copy(
                    ta.at[ia_v.at[s * KC + j]],
                    ra_v.at[pl.ds(j * CHUNK, CHUNK)], sa))
                descs.append(pltpu.async_copy(
                    tb.at[ib_v.at[s * KC + j]],
                    rb_v.at[pl.ds(j * CHUNK, CHUNK)], sb))
            for d in descs:
                d.wait()
            base = base_w + s * SUP
            wa = pltpu.async_copy(ra_v, oa.at[pl.ds(base, SUP)], sw)
            wb = pltpu.async_copy(rb_v, ob.at[pl.ds(base, SUP)], sw)
            wa.wait()
            wb.wait()
            return carry

        lax.fori_loop(0, NSUP, body, 0)

    return k


def _make_gat(F):
    @functools.partial(
        pl.kernel,
        out_type=jax.ShapeDtypeStruct((NC * (F + 1), N2), jnp.float32),
        mesh=_sc_mesh(),
        compiler_params=pltpu.CompilerParams(
            use_tc_tiling_on_sc=False, needs_layout_passes=False),
        scratch_types=[
            pltpu.VMEM((N,), jnp.float32),
            pltpu.VMEM((N,), jnp.float32),
            pltpu.VMEM((CPW, CHUNK), jnp.float32),
            pltpu.VMEM((CPW, CHUNK), jnp.int32),
            pltpu.VMEM((CPW, CHUNK), jnp.int32),
            pltpu.VMEM((2, CHUNK, F), jnp.float32),
            [pltpu.VMEM((CHUNK,), jnp.float32) for _ in range(F + 1)],
            pltpu.VMEM_SHARED((F + 1, N2), jnp.float32),
            pltpu.SemaphoreType.DMA,
            pltpu.SemaphoreType.DMA,
        ],
    )
    def k(h_tab, asrc_h, adst_h, ae_h, src_h, dst_h,
          zeros_h, out, asrc_v, adst_v, ae_v, src_v, dst_v,
          hb, cols, acc, sga, sgb):
        cid = lax.axis_index("c")
        sid = lax.axis_index("s")
        wid = sid * NC + cid
        rb = sid * RPT
        for j in range(F + 1):
            pltpu.sync_copy(zeros_h.at[pl.ds(rb, RPT)],
                            acc.at[j].at[pl.ds(rb, RPT)])
        pltpu.sync_copy(asrc_h, asrc_v)
        pltpu.sync_copy(adst_h, adst_v)
        crow = wid * CPW
        base_wk = wid * (CPW * CHUNK)
        pltpu.sync_copy(ae_h.at[pl.ds(crow, CPW)], ae_v)
        pltpu.sync_copy(src_h.at[pl.ds(crow, CPW)], src_v)
        pltpu.sync_copy(dst_h.at[pl.ds(crow, CPW)], dst_v)
        plsc.subcore_barrier()
        nch = jnp.minimum(CPW, (E - base_wk) // CHUNK)
        lane0 = lax.iota(jnp.int32, 16)
        pltpu.async_copy(h_tab.at[src_v.at[0]], hb.at[0], sga)

        def compute(c, buf, sem):
            pltpu.make_async_copy(h_tab.at[src_v.at[c]], hb.at[buf], sem).wait()
            hbb = hb.at[buf]
            cv = jnp.full((16,), 0, jnp.int32) + c
            for v in range(CHUNK // 16):
                lane16 = lane0 + v * 16
                si = plsc.load_gather(src_v, [cv, lane16])
                di = plsc.load_gather(dst_v, [cv, lane16])
                a_s = plsc.load_gather(asrc_v, [si])
                a_d = plsc.load_gather(adst_v, [di])
                t = a_s + a_d + plsc.load_gather(ae_v, [cv, lane16])
                p = jnp.exp(jnp.maximum(t, 0.2 * t))
                lane = lane0 + v * 16
                cols[F][pl.ds(v * 16, 16)] = p
                for j in range(F):
                    jv = jnp.full((16,), j, jnp.int32)
                    cj = plsc.load_gather(hbb, [lane, jv])
                    cols[j][pl.ds(v * 16, 16)] = cj * p
            for j in range(F + 1):
                pltpu.sync_copy(cols[j], acc.at[j].at[dst_v.at[c]], add=True)

        def body(i, carry):
            c0 = 2 * i
            pltpu.async_copy(h_tab.at[src_v.at[c0 + 1]], hb.at[1], sgb)
            compute(c0, 0, sga)

            @pl.when(c0 + 2 < nch)
            def _():
                pltpu.async_copy(h_tab.at[src_v.at[c0 + 2]], hb.at[0], sga)

            compute(c0 + 1, 1, sgb)
            return carry

        lax.fori_loop(0, nch // 2, body, 0)
        plsc.subcore_barrier()
        for j in range(F + 1):
            pltpu.sync_copy(acc.at[j].at[pl.ds(rb, RPT)],
                            out.at[cid * (F + 1) + j].at[pl.ds(rb, RPT)])

    return k


def _make_scatter(W):
    @functools.partial(
        pl.kernel,
        out_type=jax.ShapeDtypeStruct((NC * N2, W), jnp.float32),
        mesh=_sc_mesh(),
        compiler_params=pltpu.CompilerParams(use_tc_tiling_on_sc=False),
        scratch_types=[
            pltpu.VMEM((CPW, CHUNK), jnp.int32),
            pltpu.VMEM((SUP, W), jnp.float32),
            pltpu.VMEM_SHARED((N2, W), jnp.float32),
            pltpu.SemaphoreType.DMA,
            pltpu.SemaphoreType.DMA,
        ],
    )
    def k(pay_hbm, dst_hbm, zeros_hbm, out, idx_v, pay_v, accum, sp, ss):
        cid = lax.axis_index("c")
        sid = lax.axis_index("s")
        wid = sid * NC + cid
        rbase = sid * RPT
        pltpu.sync_copy(zeros_hbm.at[pl.ds(rbase, RPT)],
                        accum.at[pl.ds(rbase, RPT)])
        plsc.subcore_barrier()
        base_w = wid * (CPW * CHUNK)
        crow = wid * CPW
        pltpu.sync_copy(dst_hbm.at[pl.ds(crow, CPW)], idx_v)
        nsup = jnp.minimum(NSUP, (E - base_w) // SUP)

        def body(s, carry):
            base = base_w + s * SUP
            pltpu.sync_copy(pay_hbm.at[pl.ds(base, SUP)], pay_v)
            descs = []
            for j in range(KC):
                descs.append(pltpu.async_copy(
                    pay_v.at[pl.ds(j * CHUNK, CHUNK)],
                    accum.at[idx_v.at[s * KC + j]], ss, add=True))
            for d in descs:
                d.wait()
            return carry

        lax.fori_loop(0, nsup, body, 0)
        plsc.subcore_barrier()
        pltpu.sync_copy(accum.at[pl.ds(rbase, RPT)],
                        out.at[pl.ds(cid * N2 + rbase, RPT)])

    return k


def _leaky(t):
    return jnp.where(t > 0, t, 0.2 * t)


def _build_tc(interpret=False):
    tc = {}

    def a1_body(x_ref, parts_ref, w1_ref, as_ref, ad_ref, wv1_ref, wv2_ref,
                h1_ref, asrc_ref, adt_ref, aself_ref, ael2_ref):
        h1 = jnp.dot(x_ref[...], w1_ref[...], preferred_element_type=jnp.float32)
        asrc = h1 @ as_ref[...]
        adst = h1 @ ad_ref[...]
        agg = parts_ref[0:N, :] + parts_ref[N2:N2 + N, :]
        la = agg[:, 0:16] / jnp.maximum(agg[:, 16:17], 1.0)
        ael1 = la @ wv1_ref[...]
        ael2 = la @ wv2_ref[...]
        aself_ref[...] = jnp.exp(_leaky(asrc + adst + ael1))
        ael2_ref[...] = ael2
        adt_ref[...] = adst
        asrc_ref[...] = asrc
        h1_ref[...] = h1

    tc['a1'] = pl.pallas_call(
        a1_body,
        out_shape=(jax.ShapeDtypeStruct((N, 16), jnp.float32),
                   jax.ShapeDtypeStruct((N, 1), jnp.float32),
                   jax.ShapeDtypeStruct((N, 1), jnp.float32),
                   jax.ShapeDtypeStruct((N, 1), jnp.float32),
                   jax.ShapeDtypeStruct((N, 1), jnp.float32)),
        interpret=interpret)

    def a2_body(ea_ref, wv12_ref, o1_ref, o2_ref):
        ae = jnp.dot(ea_ref[...], wv12_ref[...],
                     preferred_element_type=jnp.float32)
        o1_ref[...] = jnp.reshape(ae[:, 0:1], (BLK // CHUNK, CHUNK))
        o2_ref[...] = jnp.reshape(ae[:, 1:2], (BLK // CHUNK, CHUNK))

    tc['a2'] = pl.pallas_call(
        a2_body,
        grid=(NBLK,),
        in_specs=[pl.BlockSpec((BLK, 16), lambda i: (i, 0)),
                  pl.BlockSpec((16, 2), lambda i: (0, 0))],
        out_specs=(pl.BlockSpec((BLK // CHUNK, CHUNK), lambda i: (i, 0)),
                   pl.BlockSpec((BLK // CHUNK, CHUNK), lambda i: (i, 0))),
        out_shape=(jax.ShapeDtypeStruct((EP // CHUNK, CHUNK), jnp.float32),
                   jax.ShapeDtypeStruct((EP // CHUNK, CHUNK), jnp.float32)),
        interpret=interpret)

    def c_body(agg_ref, h1_ref, aself_ref, ael2_ref, b1_ref, g_ref, bb_ref,
               w2_ref, as2_ref, ad2_ref, h2_ref, asrc2_ref, adt2_ref, aself2_ref):
        agg = agg_ref[0:N, :]
        h1 = h1_ref[...]
        aself = aself_ref[...]
        out1 = (agg[:, 0:16] + aself * h1) / (agg[:, 16:17] + aself + 1e-16)
        out1 = out1 + b1_ref[...]
        mu = jnp.mean(out1, axis=0, keepdims=True)
        var = jnp.mean((out1 - mu) ** 2, axis=0, keepdims=True)
        h1b = (out1 - mu) / jnp.sqrt(var + 1e-5) * g_ref[...] + bb_ref[...]
        h2 = jnp.dot(h1b, w2_ref[...], preferred_element_type=jnp.float32)
        asrc2 = h2 @ as2_ref[...]
        adst2 = h2 @ ad2_ref[...]
        aself2_ref[...] = jnp.exp(_leaky(asrc2 + adst2 + ael2_ref[...]))
        adt2_ref[...] = adst2
        asrc2_ref[...] = asrc2
        h2_ref[...] = h2

    tc['c'] = pl.pallas_call(
        c_body,
        out_shape=(jax.ShapeDtypeStruct((N, 32), jnp.float32),
                   jax.ShapeDtypeStruct((N, 1), jnp.float32),
                   jax.ShapeDtypeStruct((N, 1), jnp.float32),
                   jax.ShapeDtypeStruct((N, 1), jnp.float32)),
        interpret=interpret)

    def e_body(agg_ref, h2_ref, aself2_ref, b2_ref, hout_ref):
        agg = agg_ref[0:N, :]
        h2 = h2_ref[...]
        aself = aself2_ref[...]
        hout = (agg[:, 0:32] + aself * h2) / (agg[:, 32:33] + aself + 1e-16)
        hout_ref[...] = hout + b2_ref[...]

    tc['e'] = pl.pallas_call(
        e_body,
        out_shape=jax.ShapeDtypeStruct((N, 32), jnp.float32),
        interpret=interpret)

    def f_body(gs_ref, gd_ref, ea_ref, wms_ref, wmd_ref, wme_ref, bm1_ref,
               wm2_ref, bm2_ref, wdy_ref, wde_ref, bd12_ref,
               z_ref, st_ref):
        i = pl.program_id(0)
        ea = ea_ref[...]
        t = (jnp.dot(gs_ref[...], wms_ref[...], preferred_element_type=jnp.float32)
             + jnp.dot(gd_ref[...], wmd_ref[...], preferred_element_type=jnp.float32)
             + jnp.dot(ea, wme_ref[...], preferred_element_type=jnp.float32)
             + bm1_ref[...])
        y = jnp.dot(jnp.maximum(t, 0.0), wm2_ref[...],
                    preferred_element_type=jnp.float32) + bm2_ref[...]
        z = (jnp.dot(y, wdy_ref[...], preferred_element_type=jnp.float32)
             + jnp.dot(ea, wde_ref[...], preferred_element_type=jnp.float32)
             + bd12_ref[...])
        z_ref[...] = z
        rows = lax.broadcasted_iota(jnp.int32, (BLK, 1), 0) + i * BLK
        zm = jnp.where(rows < E, z, 0.0)
        st = jnp.concatenate([jnp.sum(zm, axis=0, keepdims=True),
                              jnp.sum(zm * zm, axis=0, keepdims=True)], axis=0)

        @pl.when(i == 0)
        def _():
            st_ref[...] = st

        @pl.when(i > 0)
        def _():
            st_ref[...] = st_ref[...] + st

    tc['f'] = pl.pallas_call(
        f_body,
        grid=(NBLK,),
        in_specs=[pl.BlockSpec((BLK, 32), lambda i: (i, 0)),
                  pl.BlockSpec((BLK, 32), lambda i: (i, 0)),
                  pl.BlockSpec((BLK, 16), lambda i: (i, 0)),
                  pl.BlockSpec((32, 32), lambda i: (0, 0)),
                  pl.BlockSpec((32, 32), lambda i: (0, 0)),
                  pl.BlockSpec((16, 32), lambda i: (0, 0)),
                  pl.BlockSpec((1, 32), lambda i: (0, 0)),
                  pl.BlockSpec((32, 32), lambda i: (0, 0)),
                  pl.BlockSpec((1, 32), lambda i: (0, 0)),
                  pl.BlockSpec((32, 32), lambda i: (0, 0)),
                  pl.BlockSpec((16, 32), lambda i: (0, 0)),
                  pl.BlockSpec((1, 32), lambda i: (0, 0))],
        out_specs=(pl.BlockSpec((BLK, 32), lambda i: (i, 0)),
                   pl.BlockSpec((2, 32), lambda i: (0, 0))),
        out_shape=(jax.ShapeDtypeStruct((EP, 32), jnp.float32),
                   jax.ShapeDtypeStruct((2, 32), jnp.float32)),
        interpret=interpret)

    def g_body(z_ref, zst_ref, g_ref, b_ref, wd34_ref, bd34_ref, u_ref, st_ref):
        i = pl.program_id(0)
        mu = zst_ref[0:1, :] / E
        var = zst_ref[1:2, :] / E - mu * mu
        s1 = g_ref[...] / jnp.sqrt(var + 1e-5)
        t1 = b_ref[...] - mu * s1
        zn = jnp.maximum(z_ref[...] * s1 + t1, 0.0)
        u = jnp.dot(zn, wd34_ref[...], preferred_element_type=jnp.float32) + bd34_ref[...]
        u_ref[...] = u
        rows = lax.broadcasted_iota(jnp.int32, (BLK, 1), 0) + i * BLK
        um = jnp.where(rows < E, u, 0.0)
        st = jnp.concatenate([jnp.sum(um, axis=0, keepdims=True),
                              jnp.sum(um * um, axis=0, keepdims=True)], axis=0)

        @pl.when(i == 0)
        def _():
            st_ref[...] = st

        @pl.when(i > 0)
        def _():
            st_ref[...] = st_ref[...] + st

    tc['g'] = pl.pallas_call(
        g_body,
        grid=(NBLK,),
        in_specs=[pl.BlockSpec((BLK, 32), lambda i: (i, 0)),
                  pl.BlockSpec((2, 32), lambda i: (0, 0)),
                  pl.BlockSpec((1, 32), lambda i: (0, 0)),
                  pl.BlockSpec((1, 32), lambda i: (0, 0)),
                  pl.BlockSpec((32, 16), lambda i: (0, 0)),
                  pl.BlockSpec((1, 16), lambda i: (0, 0))],
        out_specs=(pl.BlockSpec((BLK, 16), lambda i: (i, 0)),
                   pl.BlockSpec((2, 16), lambda i: (0, 0))),
        out_shape=(jax.ShapeDtypeStruct((EP, 16), jnp.float32),
                   jax.ShapeDtypeStruct((2, 16), jnp.float32)),
        interpret=interpret)

    def h_body(u_ref, ust_ref, g_ref, b_ref, wd56_ref, bd56_ref, out_ref):
        mu = ust_ref[0:1, :] / E
        var = ust_ref[1:2, :] / E - mu * mu
        s2 = g_ref[...] / jnp.sqrt(var + 1e-5)
        t2 = b_ref[...] - mu * s2
        un = jnp.maximum(u_ref[...] * s2 + t2, 0.0)
        v = jnp.dot(un, wd56_ref[...], preferred_element_type=jnp.float32) + bd56_ref[...]
        out_ref[...] = jnp.reshape(1.0 / (1.0 + jnp.exp(-v)),
                                   (BLK // CHUNK, CHUNK))

    tc['h'] = pl.pallas_call(
        h_body,
        grid=(NBLK,),
        in_specs=[pl.BlockSpec((BLK, 16), lambda i: (i, 0)),
                  pl.BlockSpec((2, 16), lambda i: (0, 0)),
                  pl.BlockSpec((1, 16), lambda i: (0, 0)),
                  pl.BlockSpec((1, 16), lambda i: (0, 0)),
                  pl.BlockSpec((16, 1), lambda i: (0, 0)),
                  pl.BlockSpec((1, 1), lambda i: (0, 0))],
        out_specs=pl.BlockSpec((BLK // CHUNK, CHUNK), lambda i: (i, 0)),
        out_shape=jax.ShapeDtypeStruct((EP // CHUNK, CHUNK), jnp.float32),
        interpret=interpret)

    return tc


_IMPL = []


def _get_impl():
    if not _IMPL:
        tc = _build_tc()
        sc = {
            'g32': _make_gather(32, 32),
            's17': _make_scatter(17),
            'gat16': _make_gat(16),
            'gat32': _make_gat(32),
        }
        _IMPL.append((tc, sc))
    return _IMPL[0]


def _pipeline(x, edge_index, edge_attr, params, tc, sc):
    p = params
    src = edge_index[0]
    dst = edge_index[1]
    pad_e = EP - E
    src_p = jnp.pad(src, (0, pad_e)).astype(jnp.int32).reshape(EP // CHUNK, CHUNK)
    dst_p = jnp.pad(dst, (0, pad_e)).astype(jnp.int32).reshape(EP // CHUNK, CHUNK)
    ea_p = jnp.pad(edge_attr, ((0, pad_e), (0, 0)))
    ea1_p = jnp.pad(jnp.concatenate(
        [edge_attr, jnp.ones((E, 1), jnp.float32)], axis=1),
        ((0, pad_e), (0, 0)))
    z17 = jnp.zeros((N2, 17), jnp.float32)
    z1d = jnp.zeros((N2,), jnp.float32)

    wv1 = (p['We1'] @ p['att_edge1']).reshape(16, 1)
    wv2 = (p['We2'] @ p['att_edge2']).reshape(16, 1)
    wv12 = jnp.concatenate([wv1, wv2], axis=1)
    wd12 = p['Wd1'] @ p['Wd2']
    bd12 = (p['bd1'] @ p['Wd2'] + p['bd2']).reshape(1, 32)
    wd34 = p['Wd3'] @ p['Wd4']
    bd34 = (p['bd3'] @ p['Wd4'] + p['bd4']).reshape(1, 16)
    wd56 = p['Wd5'] @ p['Wd6']
    bd56 = (p['bd5'] @ p['Wd6'] + p['bd6']).reshape(1, 1)

    parts0 = sc['s17'](ea1_p, dst_p, z17)

    h1, asrc1, adt1, aself1, ael2 = tc['a1'](
        x, parts0, p['W1'], p['att_src1'].reshape(16, 1),
        p['att_dst1'].reshape(16, 1), wv1, wv2)
    ae1, ae2 = tc['a2'](ea_p, wv12)

    parts1 = sc['gat16'](h1, asrc1.reshape(N), adt1.reshape(N),
                         ae1, src_p, dst_p, z1d)
    agg1 = (parts1[0:17] + parts1[17:34]).T

    h2, asrc2, adt2, aself2 = tc['c'](
        agg1, h1, aself1, ael2, p['b1'].reshape(1, 16),
        p['bn16_g'].reshape(1, 16), p['bn16_b'].reshape(1, 16),
        p['W2'], p['att_src2'].reshape(32, 1), p['att_dst2'].reshape(32, 1))

    parts2 = sc['gat32'](h2, asrc2.reshape(N), adt2.reshape(N),
                         ae2, src_p, dst_p, z1d)
    agg2 = (parts2[0:33] + parts2[33:66]).T
    hout = tc['e'](agg2, h2, aself2, p['b2'].reshape(1, 32))

    gs, gd = sc['g32'](hout, hout, src_p, dst_p)
    z, zst = tc['f'](gs, gd, ea_p,
                     p['Wm1'][0:32], p['Wm1'][32:64], p['Wm1'][64:80],
                     p['bm1'].reshape(1, 32), p['Wm2'], p['bm2'].reshape(1, 32),
                     wd12[0:32], wd12[32:48], bd12)
    u, ust = tc['g'](z, zst, p['bnd32_g'].reshape(1, 32),
                     p['bnd32_b'].reshape(1, 32), wd34, bd34)
    outp = tc['h'](u, ust, p['bnd16_g'].reshape(1, 16),
                   p['bnd16_b'].reshape(1, 16), wd56, bd56)
    return outp.reshape(EP, 1)[:E]


def kernel(x, edge_index, edge_attr, params, P, D, K):
    tc, sc = _get_impl()
    return _pipeline(x, edge_index, edge_attr, params, tc, sc)


# speedup vs baseline: 9.0801x; 1.0934x over previous
import functools

import jax
import jax.numpy as jnp
from jax import lax
from jax.experimental import pallas as pl
from jax.experimental.pallas import tpu as pltpu
from jax.experimental.pallas import tpu_sc as plsc

N = 10000
E = 320000
NC, NS = 2, 16
NW = NC * NS
CHUNK = 128
CPW = 80
EP = NW * CHUNK * CPW
N2 = 10240
RPT = N2 // NS
BLK = 2048
NBLK = EP // BLK


def _sc_mesh():
    return plsc.VectorSubcoreMesh(core_axis_name="c", subcore_axis_name="s",
                                  num_cores=NC, num_subcores=NS)


SUP = 1280
KC = SUP // CHUNK
NSUP = CPW * CHUNK // SUP


def _make_gather(WA, WB):
    @functools.partial(
        pl.kernel,
        out_type=(jax.ShapeDtypeStruct((EP, WA), jnp.float32),
                  jax.ShapeDtypeStruct((EP, WB), jnp.float32)),
        mesh=_sc_mesh(),
        compiler_params=pltpu.CompilerParams(use_tc_tiling_on_sc=False),
        scratch_types=[
            pltpu.VMEM((CPW, CHUNK), jnp.int32),
            pltpu.VMEM((CPW, CHUNK), jnp.int32),
            pltpu.VMEM((SUP, WA), jnp.float32),
            pltpu.VMEM((SUP, WB), jnp.float32),
            pltpu.SemaphoreType.DMA,
            pltpu.SemaphoreType.DMA,
            pltpu.SemaphoreType.DMA,
        ],
    )
    def k(ta, tb, ia_hbm, ib_hbm, oa, ob, ia_v, ib_v, ra_v, rb_v, sa, sb, sw):
        cid = lax.axis_index("c")
        sid = lax.axis_index("s")
        wid = sid * NC + cid
        base_w = wid * (CPW * CHUNK)
        crow = wid * CPW
        pltpu.sync_copy(ia_hbm.at[pl.ds(crow, CPW)], ia_v)
        pltpu.sync_copy(ib_hbm.at[pl.ds(crow, CPW)], ib_v)

        def body(s, carry):
            descs = []
            for j in range(KC):
                descs.append(pltpu.async_copy(
                    ta.at[ia_v.at[s * KC + j]],
                    ra_v.at[pl.ds(j * CHUNK, CHUNK)], sa))
                descs.append(pltpu.async_copy(
                    tb.at[ib_v.at[s * KC + j]],
                    rb_v.at[pl.ds(j * CHUNK, CHUNK)], sb))
            for d in descs:
                d.wait()
            base = base_w + s * SUP
            wa = pltpu.async_copy(ra_v, oa.at[pl.ds(base, SUP)], sw)
            wb = pltpu.async_copy(rb_v, ob.at[pl.ds(base, SUP)], sw)
            wa.wait()
            wb.wait()
            return carry

        lax.fori_loop(0, NSUP, body, 0)

    return k


def _make_gat(F):
    @functools.partial(
        pl.kernel,
        out_type=jax.ShapeDtypeStruct((NC * (F + 1), N2), jnp.float32),
        mesh=_sc_mesh(),
        compiler_params=pltpu.CompilerParams(
            use_tc_tiling_on_sc=False, needs_layout_passes=False),
        scratch_types=[
            pltpu.VMEM((N,), jnp.float32),
            pltpu.VMEM((N,), jnp.float32),
            pltpu.VMEM((CPW, CHUNK), jnp.float32),
            pltpu.VMEM((CPW, CHUNK), jnp.int32),
            pltpu.VMEM((CPW, CHUNK), jnp.int32),
            pltpu.VMEM((2, CHUNK, F), jnp.float32),
            [pltpu.VMEM((CHUNK,), jnp.float32) for _ in range(F + 1)],
            pltpu.VMEM_SHARED((F + 1, N2), jnp.float32),
            pltpu.SemaphoreType.DMA,
            pltpu.SemaphoreType.DMA,
            pltpu.SemaphoreType.DMA,
        ],
    )
    def k(h_tab, asrc_h, adst_h, ae_h, src_h, dst_h,
          zeros_h, out, asrc_v, adst_v, ae_v, src_v, dst_v,
          hb, cols, acc, sga, sgb, ssc):
        cid = lax.axis_index("c")
        sid = lax.axis_index("s")
        wid = sid * NC + cid
        rb = sid * RPT
        for j in range(F + 1):
            pltpu.sync_copy(zeros_h.at[pl.ds(rb, RPT)],
                            acc.at[j].at[pl.ds(rb, RPT)])
        pltpu.sync_copy(asrc_h, asrc_v)
        pltpu.sync_copy(adst_h, adst_v)
        crow = wid * CPW
        base_wk = wid * (CPW * CHUNK)
        pltpu.sync_copy(ae_h.at[pl.ds(crow, CPW)], ae_v)
        pltpu.sync_copy(src_h.at[pl.ds(crow, CPW)], src_v)
        pltpu.sync_copy(dst_h.at[pl.ds(crow, CPW)], dst_v)
        plsc.subcore_barrier()
        nch = jnp.minimum(CPW, (E - base_wk) // CHUNK)
        lane0 = lax.iota(jnp.int32, 16)
        pltpu.async_copy(h_tab.at[src_v.at[0]], hb.at[0], sga)

        def compute(c, buf, sem):
            pltpu.make_async_copy(h_tab.at[src_v.at[c]], hb.at[buf], sem).wait()
            hbb = hb.at[buf]
            cv = jnp.full((16,), 0, jnp.int32) + c
            for v in range(CHUNK // 16):
                lane16 = lane0 + v * 16
                si = plsc.load_gather(src_v, [cv, lane16])
                di = plsc.load_gather(dst_v, [cv, lane16])
                a_s = plsc.load_gather(asrc_v, [si])
                a_d = plsc.load_gather(adst_v, [di])
                t = a_s + a_d + plsc.load_gather(ae_v, [cv, lane16])
                p = jnp.exp(jnp.maximum(t, 0.2 * t))
                lane = lane0 + v * 16
                cols[F][pl.ds(v * 16, 16)] = p
                for j in range(F):
                    jv = jnp.full((16,), j, jnp.int32)
                    cj = plsc.load_gather(hbb, [lane, jv])
                    cols[j][pl.ds(v * 16, 16)] = cj * p
            descs = []
            for j in range(F + 1):
                descs.append(pltpu.async_copy(
                    cols[j], acc.at[j].at[dst_v.at[c]], ssc, add=True))
            for d in descs:
                d.wait()

        def body(i, carry):
            c0 = 2 * i
            pltpu.async_copy(h_tab.at[src_v.at[c0 + 1]], hb.at[1], sgb)
            compute(c0, 0, sga)

            @pl.when(c0 + 2 < nch)
            def _():
                pltpu.async_copy(h_tab.at[src_v.at[c0 + 2]], hb.at[0], sga)

            compute(c0 + 1, 1, sgb)
            return carry

        lax.fori_loop(0, nch // 2, body, 0)
        plsc.subcore_barrier()
        for j in range(F + 1):
            pltpu.sync_copy(acc.at[j].at[pl.ds(rb, RPT)],
                            out.at[cid * (F + 1) + j].at[pl.ds(rb, RPT)])

    return k


def _make_scatter(W):
    @functools.partial(
        pl.kernel,
        out_type=jax.ShapeDtypeStruct((NC * N2, W), jnp.float32),
        mesh=_sc_mesh(),
        compiler_params=pltpu.CompilerParams(use_tc_tiling_on_sc=False),
        scratch_types=[
            pltpu.VMEM((CPW, CHUNK), jnp.int32),
            pltpu.VMEM((SUP, W), jnp.float32),
            pltpu.VMEM_SHARED((N2, W), jnp.float32),
            pltpu.SemaphoreType.DMA,
            pltpu.SemaphoreType.DMA,
        ],
    )
    def k(pay_hbm, dst_hbm, zeros_hbm, out, idx_v, pay_v, accum, sp, ss):
        cid = lax.axis_index("c")
        sid = lax.axis_index("s")
        wid = sid * NC + cid
        rbase = sid * RPT
        pltpu.sync_copy(zeros_hbm.at[pl.ds(rbase, RPT)],
                        accum.at[pl.ds(rbase, RPT)])
        plsc.subcore_barrier()
        base_w = wid * (CPW * CHUNK)
        crow = wid * CPW
        pltpu.sync_copy(dst_hbm.at[pl.ds(crow, CPW)], idx_v)
        nsup = jnp.minimum(NSUP, (E - base_w) // SUP)

        def body(s, carry):
            base = base_w + s * SUP
            pltpu.sync_copy(pay_hbm.at[pl.ds(base, SUP)], pay_v)
            descs = []
            for j in range(KC):
                descs.append(pltpu.async_copy(
                    pay_v.at[pl.ds(j * CHUNK, CHUNK)],
                    accum.at[idx_v.at[s * KC + j]], ss, add=True))
            for d in descs:
                d.wait()
            return carry

        lax.fori_loop(0, nsup, body, 0)
        plsc.subcore_barrier()
        pltpu.sync_copy(accum.at[pl.ds(rbase, RPT)],
                        out.at[pl.ds(cid * N2 + rbase, RPT)])

    return k


def _leaky(t):
    return jnp.where(t > 0, t, 0.2 * t)


def _build_tc(interpret=False):
    tc = {}

    def a1_body(x_ref, parts_ref, w1_ref, as_ref, ad_ref, wv1_ref, wv2_ref,
                h1_ref, asrc_ref, adt_ref, aself_ref, ael2_ref):
        h1 = jnp.dot(x_ref[...], w1_ref[...], preferred_element_type=jnp.float32)
        asrc = h1 @ as_ref[...]
        adst = h1 @ ad_ref[...]
        agg = parts_ref[0:N, :] + parts_ref[N2:N2 + N, :]
        la = agg[:, 0:16] / jnp.maximum(agg[:, 16:17], 1.0)
        ael1 = la @ wv1_ref[...]
        ael2 = la @ wv2_ref[...]
        aself_ref[...] = jnp.exp(_leaky(asrc + adst + ael1))
        ael2_ref[...] = ael2
        adt_ref[...] = adst
        asrc_ref[...] = asrc
        h1_ref[...] = h1

    tc['a1'] = pl.pallas_call(
        a1_body,
        out_shape=(jax.ShapeDtypeStruct((N, 16), jnp.float32),
                   jax.ShapeDtypeStruct((N, 1), jnp.float32),
                   jax.ShapeDtypeStruct((N, 1), jnp.float32),
                   jax.ShapeDtypeStruct((N, 1), jnp.float32),
                   jax.ShapeDtypeStruct((N, 1), jnp.float32)),
        interpret=interpret)

    def a2_body(ea_ref, wv12_ref, o1_ref, o2_ref):
        ae = jnp.dot(ea_ref[...], wv12_ref[...],
                     preferred_element_type=jnp.float32)
        o1_ref[...] = jnp.reshape(ae[:, 0:1], (BLK // CHUNK, CHUNK))
        o2_ref[...] = jnp.reshape(ae[:, 1:2], (BLK // CHUNK, CHUNK))

    tc['a2'] = pl.pallas_call(
        a2_body,
        grid=(NBLK,),
        in_specs=[pl.BlockSpec((BLK, 16), lambda i: (i, 0)),
                  pl.BlockSpec((16, 2), lambda i: (0, 0))],
        out_specs=(pl.BlockSpec((BLK // CHUNK, CHUNK), lambda i: (i, 0)),
                   pl.BlockSpec((BLK // CHUNK, CHUNK), lambda i: (i, 0))),
        out_shape=(jax.ShapeDtypeStruct((EP // CHUNK, CHUNK), jnp.float32),
                   jax.ShapeDtypeStruct((EP // CHUNK, CHUNK), jnp.float32)),
        interpret=interpret)

    def c_body(agg_ref, h1_ref, aself_ref, ael2_ref, b1_ref, g_ref, bb_ref,
               w2_ref, as2_ref, ad2_ref, h2_ref, asrc2_ref, adt2_ref, aself2_ref):
        agg = agg_ref[0:N, :]
        h1 = h1_ref[...]
        aself = aself_ref[...]
        out1 = (agg[:, 0:16] + aself * h1) / (agg[:, 16:17] + aself + 1e-16)
        out1 = out1 + b1_ref[...]
        mu = jnp.mean(out1, axis=0, keepdims=True)
        var = jnp.mean((out1 - mu) ** 2, axis=0, keepdims=True)
        h1b = (out1 - mu) / jnp.sqrt(var + 1e-5) * g_ref[...] + bb_ref[...]
        h2 = jnp.dot(h1b, w2_ref[...], preferred_element_type=jnp.float32)
        asrc2 = h2 @ as2_ref[...]
        adst2 = h2 @ ad2_ref[...]
        aself2_ref[...] = jnp.exp(_leaky(asrc2 + adst2 + ael2_ref[...]))
        adt2_ref[...] = adst2
        asrc2_ref[...] = asrc2
        h2_ref[...] = h2

    tc['c'] = pl.pallas_call(
        c_body,
        out_shape=(jax.ShapeDtypeStruct((N, 32), jnp.float32),
                   jax.ShapeDtypeStruct((N, 1), jnp.float32),
                   jax.ShapeDtypeStruct((N, 1), jnp.float32),
                   jax.ShapeDtypeStruct((N, 1), jnp.float32)),
        interpret=interpret)

    def e_body(agg_ref, h2_ref, aself2_ref, b2_ref, hout_ref):
        agg = agg_ref[0:N, :]
        h2 = h2_ref[...]
        aself = aself2_ref[...]
        hout = (agg[:, 0:32] + aself * h2) / (agg[:, 32:33] + aself + 1e-16)
        hout_ref[...] = hout + b2_ref[...]

    tc['e'] = pl.pallas_call(
        e_body,
        out_shape=jax.ShapeDtypeStruct((N, 32), jnp.float32),
        interpret=interpret)

    def f_body(gs_ref, gd_ref, ea_ref, wms_ref, wmd_ref, wme_ref, bm1_ref,
               wm2_ref, bm2_ref, wdy_ref, wde_ref, bd12_ref,
               z_ref, st_ref):
        i = pl.program_id(0)
        ea = ea_ref[...]
        t = (jnp.dot(gs_ref[...], wms_ref[...], preferred_element_type=jnp.float32)
             + jnp.dot(gd_ref[...], wmd_ref[...], preferred_element_type=jnp.float32)
             + jnp.dot(ea, wme_ref[...], preferred_element_type=jnp.float32)
             + bm1_ref[...])
        y = jnp.dot(jnp.maximum(t, 0.0), wm2_ref[...],
                    preferred_element_type=jnp.float32) + bm2_ref[...]
        z = (jnp.dot(y, wdy_ref[...], preferred_element_type=jnp.float32)
             + jnp.dot(ea, wde_ref[...], preferred_element_type=jnp.float32)
             + bd12_ref[...])
        z_ref[...] = z
        rows = lax.broadcasted_iota(jnp.int32, (BLK, 1), 0) + i * BLK
        zm = jnp.where(rows < E, z, 0.0)
        st = jnp.concatenate([jnp.sum(zm, axis=0, keepdims=True),
                              jnp.sum(zm * zm, axis=0, keepdims=True)], axis=0)

        @pl.when(i == 0)
        def _():
            st_ref[...] = st

        @pl.when(i > 0)
        def _():
            st_ref[...] = st_ref[...] + st

    tc['f'] = pl.pallas_call(
        f_body,
        grid=(NBLK,),
        in_specs=[pl.BlockSpec((BLK, 32), lambda i: (i, 0)),
                  pl.BlockSpec((BLK, 32), lambda i: (i, 0)),
                  pl.BlockSpec((BLK, 16), lambda i: (i, 0)),
                  pl.BlockSpec((32, 32), lambda i: (0, 0)),
                  pl.BlockSpec((32, 32), lambda i: (0, 0)),
                  pl.BlockSpec((16, 32), lambda i: (0, 0)),
                  pl.BlockSpec((1, 32), lambda i: (0, 0)),
                  pl.BlockSpec((32, 32), lambda i: (0, 0)),
                  pl.BlockSpec((1, 32), lambda i: (0, 0)),
                  pl.BlockSpec((32, 32), lambda i: (0, 0)),
                  pl.BlockSpec((16, 32), lambda i: (0, 0)),
                  pl.BlockSpec((1, 32), lambda i: (0, 0))],
        out_specs=(pl.BlockSpec((BLK, 32), lambda i: (i, 0)),
                   pl.BlockSpec((2, 32), lambda i: (0, 0))),
        out_shape=(jax.ShapeDtypeStruct((EP, 32), jnp.float32),
                   jax.ShapeDtypeStruct((2, 32), jnp.float32)),
        interpret=interpret)

    def g_body(z_ref, zst_ref, g_ref, b_ref, wd34_ref, bd34_ref, u_ref, st_ref):
        i = pl.program_id(0)
        mu = zst_ref[0:1, :] / E
        var = zst_ref[1:2, :] / E - mu * mu
        s1 = g_ref[...] / jnp.sqrt(var + 1e-5)
        t1 = b_ref[...] - mu * s1
        zn = jnp.maximum(z_ref[...] * s1 + t1, 0.0)
        u = jnp.dot(zn, wd34_ref[...], preferred_element_type=jnp.float32) + bd34_ref[...]
        u_ref[...] = u
        rows = lax.broadcasted_iota(jnp.int32, (BLK, 1), 0) + i * BLK
        um = jnp.where(rows < E, u, 0.0)
        st = jnp.concatenate([jnp.sum(um, axis=0, keepdims=True),
                              jnp.sum(um * um, axis=0, keepdims=True)], axis=0)

        @pl.when(i == 0)
        def _():
            st_ref[...] = st

        @pl.when(i > 0)
        def _():
            st_ref[...] = st_ref[...] + st

    tc['g'] = pl.pallas_call(
        g_body,
        grid=(NBLK,),
        in_specs=[pl.BlockSpec((BLK, 32), lambda i: (i, 0)),
                  pl.BlockSpec((2, 32), lambda i: (0, 0)),
                  pl.BlockSpec((1, 32), lambda i: (0, 0)),
                  pl.BlockSpec((1, 32), lambda i: (0, 0)),
                  pl.BlockSpec((32, 16), lambda i: (0, 0)),
                  pl.BlockSpec((1, 16), lambda i: (0, 0))],
        out_specs=(pl.BlockSpec((BLK, 16), lambda i: (i, 0)),
                   pl.BlockSpec((2, 16), lambda i: (0, 0))),
        out_shape=(jax.ShapeDtypeStruct((EP, 16), jnp.float32),
                   jax.ShapeDtypeStruct((2, 16), jnp.float32)),
        interpret=interpret)

    def h_body(u_ref, ust_ref, g_ref, b_ref, wd56_ref, bd56_ref, out_ref):
        mu = ust_ref[0:1, :] / E
        var = ust_ref[1:2, :] / E - mu * mu
        s2 = g_ref[...] / jnp.sqrt(var + 1e-5)
        t2 = b_ref[...] - mu * s2
        un = jnp.maximum(u_ref[...] * s2 + t2, 0.0)
        v = jnp.dot(un, wd56_ref[...], preferred_element_type=jnp.float32) + bd56_ref[...]
        out_ref[...] = jnp.reshape(1.0 / (1.0 + jnp.exp(-v)),
                                   (BLK // CHUNK, CHUNK))

    tc['h'] = pl.pallas_call(
        h_body,
        grid=(NBLK,),
        in_specs=[pl.BlockSpec((BLK, 16), lambda i: (i, 0)),
                  pl.BlockSpec((2, 16), lambda i: (0, 0)),
                  pl.BlockSpec((1, 16), lambda i: (0, 0)),
                  pl.BlockSpec((1, 16), lambda i: (0, 0)),
                  pl.BlockSpec((16, 1), lambda i: (0, 0)),
                  pl.BlockSpec((1, 1), lambda i: (0, 0))],
        out_specs=pl.BlockSpec((BLK // CHUNK, CHUNK), lambda i: (i, 0)),
        out_shape=jax.ShapeDtypeStruct((EP // CHUNK, CHUNK), jnp.float32),
        interpret=interpret)

    return tc


_IMPL = []


def _get_impl():
    if not _IMPL:
        tc = _build_tc()
        sc = {
            'g32': _make_gather(32, 32),
            's17': _make_scatter(17),
            'gat16': _make_gat(16),
            'gat32': _make_gat(32),
        }
        _IMPL.append((tc, sc))
    return _IMPL[0]


def _pipeline(x, edge_index, edge_attr, params, tc, sc):
    p = params
    src = edge_index[0]
    dst = edge_index[1]
    pad_e = EP - E
    src_p = jnp.pad(src, (0, pad_e)).astype(jnp.int32).reshape(EP // CHUNK, CHUNK)
    dst_p = jnp.pad(dst, (0, pad_e)).astype(jnp.int32).reshape(EP // CHUNK, CHUNK)
    ea_p = jnp.pad(edge_attr, ((0, pad_e), (0, 0)))
    ea1_p = jnp.pad(jnp.concatenate(
        [edge_attr, jnp.ones((E, 1), jnp.float32)], axis=1),
        ((0, pad_e), (0, 0)))
    z17 = jnp.zeros((N2, 17), jnp.float32)
    z1d = jnp.zeros((N2,), jnp.float32)

    wv1 = (p['We1'] @ p['att_edge1']).reshape(16, 1)
    wv2 = (p['We2'] @ p['att_edge2']).reshape(16, 1)
    wv12 = jnp.concatenate([wv1, wv2], axis=1)
    wd12 = p['Wd1'] @ p['Wd2']
    bd12 = (p['bd1'] @ p['Wd2'] + p['bd2']).reshape(1, 32)
    wd34 = p['Wd3'] @ p['Wd4']
    bd34 = (p['bd3'] @ p['Wd4'] + p['bd4']).reshape(1, 16)
    wd56 = p['Wd5'] @ p['Wd6']
    bd56 = (p['bd5'] @ p['Wd6'] + p['bd6']).reshape(1, 1)

    parts0 = sc['s17'](ea1_p, dst_p, z17)

    h1, asrc1, adt1, aself1, ael2 = tc['a1'](
        x, parts0, p['W1'], p['att_src1'].reshape(16, 1),
        p['att_dst1'].reshape(16, 1), wv1, wv2)
    ae1, ae2 = tc['a2'](ea_p, wv12)

    parts1 = sc['gat16'](h1, asrc1.reshape(N), adt1.reshape(N),
                         ae1, src_p, dst_p, z1d)
    agg1 = (parts1[0:17] + parts1[17:34]).T

    h2, asrc2, adt2, aself2 = tc['c'](
        agg1, h1, aself1, ael2, p['b1'].reshape(1, 16),
        p['bn16_g'].reshape(1, 16), p['bn16_b'].reshape(1, 16),
        p['W2'], p['att_src2'].reshape(32, 1), p['att_dst2'].reshape(32, 1))

    parts2 = sc['gat32'](h2, asrc2.reshape(N), adt2.reshape(N),
                         ae2, src_p, dst_p, z1d)
    agg2 = (parts2[0:33] + parts2[33:66]).T
    hout = tc['e'](agg2, h2, aself2, p['b2'].reshape(1, 32))

    gs, gd = sc['g32'](hout, hout, src_p, dst_p)
    z, zst = tc['f'](gs, gd, ea_p,
                     p['Wm1'][0:32], p['Wm1'][32:64], p['Wm1'][64:80],
                     p['bm1'].reshape(1, 32), p['Wm2'], p['bm2'].reshape(1, 32),
                     wd12[0:32], wd12[32:48], bd12)
    u, ust = tc['g'](z, zst, p['bnd32_g'].reshape(1, 32),
                     p['bnd32_b'].reshape(1, 32), wd34, bd34)
    outp = tc['h'](u, ust, p['bnd16_g'].reshape(1, 16),
                   p['bnd16_b'].reshape(1, 16), wd56, bd56)
    return outp.reshape(EP, 1)[:E]


def kernel(x, edge_index, edge_attr, params, P, D, K):
    tc, sc = _get_impl()
    return _pipeline(x, edge_index, edge_attr, params, tc, sc)


# speedup vs baseline: 9.0821x; 1.0002x over previous
import functools

import jax
import jax.numpy as jnp
from jax import lax
from jax.experimental import pallas as pl
from jax.experimental.pallas import tpu as pltpu
from jax.experimental.pallas import tpu_sc as plsc

N = 10000
E = 320000
NC, NS = 2, 16
NW = NC * NS
CHUNK = 128
CPW = 80
EP = NW * CHUNK * CPW
N2 = 10240
RPT = N2 // NS
BLK = 2048
NBLK = EP // BLK


def _sc_mesh():
    return plsc.VectorSubcoreMesh(core_axis_name="c", subcore_axis_name="s",
                                  num_cores=NC, num_subcores=NS)


SUP = 1280
KC = SUP // CHUNK
NSUP = CPW * CHUNK // SUP


def _make_gather(WA, WB):
    @functools.partial(
        pl.kernel,
        out_type=(jax.ShapeDtypeStruct((EP, WA), jnp.float32),
                  jax.ShapeDtypeStruct((EP, WB), jnp.float32)),
        mesh=_sc_mesh(),
        compiler_params=pltpu.CompilerParams(use_tc_tiling_on_sc=False),
        scratch_types=[
            pltpu.VMEM((CPW, CHUNK), jnp.int32),
            pltpu.VMEM((CPW, CHUNK), jnp.int32),
            pltpu.VMEM((SUP, WA), jnp.float32),
            pltpu.VMEM((SUP, WB), jnp.float32),
            pltpu.SemaphoreType.DMA,
            pltpu.SemaphoreType.DMA,
            pltpu.SemaphoreType.DMA,
        ],
    )
    def k(ta, tb, ia_hbm, ib_hbm, oa, ob, ia_v, ib_v, ra_v, rb_v, sa, sb, sw):
        cid = lax.axis_index("c")
        sid = lax.axis_index("s")
        wid = sid * NC + cid
        base_w = wid * (CPW * CHUNK)
        crow = wid * CPW
        pltpu.sync_copy(ia_hbm.at[pl.ds(crow, CPW)], ia_v)
        pltpu.sync_copy(ib_hbm.at[pl.ds(crow, CPW)], ib_v)

        def body(s, carry):
            descs = []
            for j in range(KC):
                descs.append(pltpu.async_copy(
                    ta.at[ia_v.at[s * KC + j]],
                    ra_v.at[pl.ds(j * CHUNK, CHUNK)], sa))
                descs.append(pltpu.async_copy(
                    tb.at[ib_v.at[s * KC + j]],
                    rb_v.at[pl.ds(j * CHUNK, CHUNK)], sb))
            for d in descs:
                d.wait()
            base = base_w + s * SUP
            wa = pltpu.async_copy(ra_v, oa.at[pl.ds(base, SUP)], sw)
            wb = pltpu.async_copy(rb_v, ob.at[pl.ds(base, SUP)], sw)
            wa.wait()
            wb.wait()
            return carry

        lax.fori_loop(0, NSUP, body, 0)

    return k


def _make_gat(F):
    @functools.partial(
        pl.kernel,
        out_type=jax.ShapeDtypeStruct((NC * (F + 1), N2), jnp.float32),
        mesh=_sc_mesh(),
        compiler_params=pltpu.CompilerParams(
            use_tc_tiling_on_sc=False, needs_layout_passes=False),
        scratch_types=[
            pltpu.VMEM((N,), jnp.float32),
            pltpu.VMEM((N,), jnp.float32),
            pltpu.VMEM((CPW, CHUNK), jnp.float32),
            pltpu.VMEM((CPW, CHUNK), jnp.int32),
            pltpu.VMEM((CPW, CHUNK), jnp.int32),
            pltpu.VMEM((2, CHUNK, F), jnp.float32),
            [pltpu.VMEM((CHUNK,), jnp.float32) for _ in range(F + 1)],
            pltpu.VMEM_SHARED((F + 1, N2), jnp.float32),
            pltpu.SemaphoreType.DMA,
            pltpu.SemaphoreType.DMA,
            pltpu.SemaphoreType.DMA,
        ],
    )
    def k(h_tab, asrc_h, adst_h, ae_h, src_h, dst_h,
          zeros_h, out, asrc_v, adst_v, ae_v, src_v, dst_v,
          hb, cols, acc, sga, sgb, ssc):
        cid = lax.axis_index("c")
        sid = lax.axis_index("s")
        wid = sid * NC + cid
        rb = sid * RPT
        for j in range(F + 1):
            pltpu.sync_copy(zeros_h.at[pl.ds(rb, RPT)],
                            acc.at[j].at[pl.ds(rb, RPT)])
        pltpu.sync_copy(asrc_h, asrc_v)
        pltpu.sync_copy(adst_h, adst_v)
        crow = wid * CPW
        base_wk = wid * (CPW * CHUNK)
        pltpu.sync_copy(ae_h.at[pl.ds(crow, CPW)], ae_v)
        pltpu.sync_copy(src_h.at[pl.ds(crow, CPW)], src_v)
        pltpu.sync_copy(dst_h.at[pl.ds(crow, CPW)], dst_v)
        plsc.subcore_barrier()
        nch = jnp.minimum(CPW, (E - base_wk) // CHUNK)
        lane0 = lax.iota(jnp.int32, 16)
        pltpu.async_copy(h_tab.at[src_v.at[0]], hb.at[0], sga)

        def compute(c, buf, sem):
            pltpu.make_async_copy(h_tab.at[src_v.at[c]], hb.at[buf], sem).wait()
            hbb = hb.at[buf]
            cv = jnp.full((16,), 0, jnp.int32) + c
            for v in range(CHUNK // 16):
                lane16 = lane0 + v * 16
                si = plsc.load_gather(src_v, [cv, lane16])
                di = plsc.load_gather(dst_v, [cv, lane16])
                a_s = plsc.load_gather(asrc_v, [si])
                a_d = plsc.load_gather(adst_v, [di])
                t = a_s + a_d + plsc.load_gather(ae_v, [cv, lane16])
                p = jnp.exp(jnp.maximum(t, 0.2 * t))
                cols[F][pl.ds(v * 16, 16)] = p
                for j in range(F):
                    jv = jnp.full((16,), j, jnp.int32)
                    cj = plsc.load_gather(hbb, [lane16, jv])
                    cols[j][pl.ds(v * 16, 16)] = cj * p
            descs = []
            for j in range(F + 1):
                descs.append(pltpu.async_copy(
                    cols[j], acc.at[j].at[dst_v.at[c]], ssc, add=True))
            for d in descs:
                d.wait()

        def body(i, carry):
            c0 = 2 * i
            pltpu.async_copy(h_tab.at[src_v.at[c0 + 1]], hb.at[1], sgb)
            compute(c0, 0, sga)

            @pl.when(c0 + 2 < nch)
            def _():
                pltpu.async_copy(h_tab.at[src_v.at[c0 + 2]], hb.at[0], sga)

            compute(c0 + 1, 1, sgb)
            return carry

        lax.fori_loop(0, nch // 2, body, 0)
        plsc.subcore_barrier()
        for j in range(F + 1):
            pltpu.sync_copy(acc.at[j].at[pl.ds(rb, RPT)],
                            out.at[cid * (F + 1) + j].at[pl.ds(rb, RPT)])

    return k


def _make_scatter(W):
    @functools.partial(
        pl.kernel,
        out_type=jax.ShapeDtypeStruct((NC * N2, W), jnp.float32),
        mesh=_sc_mesh(),
        compiler_params=pltpu.CompilerParams(use_tc_tiling_on_sc=False),
        scratch_types=[
            pltpu.VMEM((CPW, CHUNK), jnp.int32),
            pltpu.VMEM((SUP, W), jnp.float32),
            pltpu.VMEM_SHARED((N2, W), jnp.float32),
            pltpu.SemaphoreType.DMA,
            pltpu.SemaphoreType.DMA,
        ],
    )
    def k(pay_hbm, dst_hbm, zeros_hbm, out, idx_v, pay_v, accum, sp, ss):
        cid = lax.axis_index("c")
        sid = lax.axis_index("s")
        wid = sid * NC + cid
        rbase = sid * RPT
        pltpu.sync_copy(zeros_hbm.at[pl.ds(rbase, RPT)],
                        accum.at[pl.ds(rbase, RPT)])
        plsc.subcore_barrier()
        base_w = wid * (CPW * CHUNK)
        crow = wid * CPW
        pltpu.sync_copy(dst_hbm.at[pl.ds(crow, CPW)], idx_v)
        nsup = jnp.minimum(NSUP, (E - base_w) // SUP)

        def body(s, carry):
            base = base_w + s * SUP
            pltpu.sync_copy(pay_hbm.at[pl.ds(base, SUP)], pay_v)
            descs = []
            for j in range(KC):
                descs.append(pltpu.async_copy(
                    pay_v.at[pl.ds(j * CHUNK, CHUNK)],
                    accum.at[idx_v.at[s * KC + j]], ss, add=True))
            for d in descs:
                d.wait()
            return carry

        lax.fori_loop(0, nsup, body, 0)
        plsc.subcore_barrier()
        pltpu.sync_copy(accum.at[pl.ds(rbase, RPT)],
                        out.at[pl.ds(cid * N2 + rbase, RPT)])

    return k


def _leaky(t):
    return jnp.where(t > 0, t, 0.2 * t)


def _build_tc(interpret=False):
    tc = {}

    def a1_body(x_ref, parts_ref, w1_ref, as_ref, ad_ref, wv1_ref, wv2_ref,
                h1_ref, asrc_ref, adt_ref, aself_ref, ael2_ref):
        h1 = jnp.dot(x_ref[...], w1_ref[...], preferred_element_type=jnp.float32)
        asrc = h1 @ as_ref[...]
        adst = h1 @ ad_ref[...]
        agg = parts_ref[0:N, :] + parts_ref[N2:N2 + N, :]
        la = agg[:, 0:16] / jnp.maximum(agg[:, 16:17], 1.0)
        ael1 = la @ wv1_ref[...]
        ael2 = la @ wv2_ref[...]
        aself_ref[...] = jnp.exp(_leaky(asrc + adst + ael1))
        ael2_ref[...] = ael2
        adt_ref[...] = adst
        asrc_ref[...] = asrc
        h1_ref[...] = h1

    tc['a1'] = pl.pallas_call(
        a1_body,
        out_shape=(jax.ShapeDtypeStruct((N, 16), jnp.float32),
                   jax.ShapeDtypeStruct((N, 1), jnp.float32),
                   jax.ShapeDtypeStruct((N, 1), jnp.float32),
                   jax.ShapeDtypeStruct((N, 1), jnp.float32),
                   jax.ShapeDtypeStruct((N, 1), jnp.float32)),
        interpret=interpret)

    def a2_body(ea_ref, wv12_ref, o1_ref, o2_ref):
        ae = jnp.dot(ea_ref[...], wv12_ref[...],
                     preferred_element_type=jnp.float32)
        o1_ref[...] = jnp.reshape(ae[:, 0:1], (BLK // CHUNK, CHUNK))
        o2_ref[...] = jnp.reshape(ae[:, 1:2], (BLK // CHUNK, CHUNK))

    tc['a2'] = pl.pallas_call(
        a2_body,
        grid=(NBLK,),
        in_specs=[pl.BlockSpec((BLK, 16), lambda i: (i, 0)),
                  pl.BlockSpec((16, 2), lambda i: (0, 0))],
        out_specs=(pl.BlockSpec((BLK // CHUNK, CHUNK), lambda i: (i, 0)),
                   pl.BlockSpec((BLK // CHUNK, CHUNK), lambda i: (i, 0))),
        out_shape=(jax.ShapeDtypeStruct((EP // CHUNK, CHUNK), jnp.float32),
                   jax.ShapeDtypeStruct((EP // CHUNK, CHUNK), jnp.float32)),
        interpret=interpret)

    def c_body(agg_ref, h1_ref, aself_ref, ael2_ref, b1_ref, g_ref, bb_ref,
               w2_ref, as2_ref, ad2_ref, h2_ref, asrc2_ref, adt2_ref, aself2_ref):
        agg = agg_ref[0:N, :]
        h1 = h1_ref[...]
        aself = aself_ref[...]
        out1 = (agg[:, 0:16] + aself * h1) / (agg[:, 16:17] + aself + 1e-16)
        out1 = out1 + b1_ref[...]
        mu = jnp.mean(out1, axis=0, keepdims=True)
        var = jnp.mean((out1 - mu) ** 2, axis=0, keepdims=True)
        h1b = (out1 - mu) / jnp.sqrt(var + 1e-5) * g_ref[...] + bb_ref[...]
        h2 = jnp.dot(h1b, w2_ref[...], preferred_element_type=jnp.float32)
        asrc2 = h2 @ as2_ref[...]
        adst2 = h2 @ ad2_ref[...]
        aself2_ref[...] = jnp.exp(_leaky(asrc2 + adst2 + ael2_ref[...]))
        adt2_ref[...] = adst2
        asrc2_ref[...] = asrc2
        h2_ref[...] = h2

    tc['c'] = pl.pallas_call(
        c_body,
        out_shape=(jax.ShapeDtypeStruct((N, 32), jnp.float32),
                   jax.ShapeDtypeStruct((N, 1), jnp.float32),
                   jax.ShapeDtypeStruct((N, 1), jnp.float32),
                   jax.ShapeDtypeStruct((N, 1), jnp.float32)),
        interpret=interpret)

    def e_body(agg_ref, h2_ref, aself2_ref, b2_ref, hout_ref):
        agg = agg_ref[0:N, :]
        h2 = h2_ref[...]
        aself = aself2_ref[...]
        hout = (agg[:, 0:32] + aself * h2) / (agg[:, 32:33] + aself + 1e-16)
        hout_ref[...] = hout + b2_ref[...]

    tc['e'] = pl.pallas_call(
        e_body,
        out_shape=jax.ShapeDtypeStruct((N, 32), jnp.float32),
        interpret=interpret)

    def f_body(gs_ref, gd_ref, ea_ref, wms_ref, wmd_ref, wme_ref, bm1_ref,
               wm2_ref, bm2_ref, wdy_ref, wde_ref, bd12_ref,
               z_ref, st_ref):
        i = pl.program_id(0)
        ea = ea_ref[...]
        t = (jnp.dot(gs_ref[...], wms_ref[...], preferred_element_type=jnp.float32)
             + jnp.dot(gd_ref[...], wmd_ref[...], preferred_element_type=jnp.float32)
             + jnp.dot(ea, wme_ref[...], preferred_element_type=jnp.float32)
             + bm1_ref[...])
        y = jnp.dot(jnp.maximum(t, 0.0), wm2_ref[...],
                    preferred_element_type=jnp.float32) + bm2_ref[...]
        z = (jnp.dot(y, wdy_ref[...], preferred_element_type=jnp.float32)
             + jnp.dot(ea, wde_ref[...], preferred_element_type=jnp.float32)
             + bd12_ref[...])
        z_ref[...] = z
        rows = lax.broadcasted_iota(jnp.int32, (BLK, 1), 0) + i * BLK
        zm = jnp.where(rows < E, z, 0.0)
        st = jnp.concatenate([jnp.sum(zm, axis=0, keepdims=True),
                              jnp.sum(zm * zm, axis=0, keepdims=True)], axis=0)

        @pl.when(i == 0)
        def _():
            st_ref[...] = st

        @pl.when(i > 0)
        def _():
            st_ref[...] = st_ref[...] + st

    tc['f'] = pl.pallas_call(
        f_body,
        grid=(NBLK,),
        in_specs=[pl.BlockSpec((BLK, 32), lambda i: (i, 0)),
                  pl.BlockSpec((BLK, 32), lambda i: (i, 0)),
                  pl.BlockSpec((BLK, 16), lambda i: (i, 0)),
                  pl.BlockSpec((32, 32), lambda i: (0, 0)),
                  pl.BlockSpec((32, 32), lambda i: (0, 0)),
                  pl.BlockSpec((16, 32), lambda i: (0, 0)),
                  pl.BlockSpec((1, 32), lambda i: (0, 0)),
                  pl.BlockSpec((32, 32), lambda i: (0, 0)),
                  pl.BlockSpec((1, 32), lambda i: (0, 0)),
                  pl.BlockSpec((32, 32), lambda i: (0, 0)),
                  pl.BlockSpec((16, 32), lambda i: (0, 0)),
                  pl.BlockSpec((1, 32), lambda i: (0, 0))],
        out_specs=(pl.BlockSpec((BLK, 32), lambda i: (i, 0)),
                   pl.BlockSpec((2, 32), lambda i: (0, 0))),
        out_shape=(jax.ShapeDtypeStruct((EP, 32), jnp.float32),
                   jax.ShapeDtypeStruct((2, 32), jnp.float32)),
        interpret=interpret)

    def g_body(z_ref, zst_ref, g_ref, b_ref, wd34_ref, bd34_ref, u_ref, st_ref):
        i = pl.program_id(0)
        mu = zst_ref[0:1, :] / E
        var = zst_ref[1:2, :] / E - mu * mu
        s1 = g_ref[...] / jnp.sqrt(var + 1e-5)
        t1 = b_ref[...] - mu * s1
        zn = jnp.maximum(z_ref[...] * s1 + t1, 0.0)
        u = jnp.dot(zn, wd34_ref[...], preferred_element_type=jnp.float32) + bd34_ref[...]
        u_ref[...] = u
        rows = lax.broadcasted_iota(jnp.int32, (BLK, 1), 0) + i * BLK
        um = jnp.where(rows < E, u, 0.0)
        st = jnp.concatenate([jnp.sum(um, axis=0, keepdims=True),
                              jnp.sum(um * um, axis=0, keepdims=True)], axis=0)

        @pl.when(i == 0)
        def _():
            st_ref[...] = st

        @pl.when(i > 0)
        def _():
            st_ref[...] = st_ref[...] + st

    tc['g'] = pl.pallas_call(
        g_body,
        grid=(NBLK,),
        in_specs=[pl.BlockSpec((BLK, 32), lambda i: (i, 0)),
                  pl.BlockSpec((2, 32), lambda i: (0, 0)),
                  pl.BlockSpec((1, 32), lambda i: (0, 0)),
                  pl.BlockSpec((1, 32), lambda i: (0, 0)),
                  pl.BlockSpec((32, 16), lambda i: (0, 0)),
                  pl.BlockSpec((1, 16), lambda i: (0, 0))],
        out_specs=(pl.BlockSpec((BLK, 16), lambda i: (i, 0)),
                   pl.BlockSpec((2, 16), lambda i: (0, 0))),
        out_shape=(jax.ShapeDtypeStruct((EP, 16), jnp.float32),
                   jax.ShapeDtypeStruct((2, 16), jnp.float32)),
        interpret=interpret)

    def h_body(u_ref, ust_ref, g_ref, b_ref, wd56_ref, bd56_ref, out_ref):
        mu = ust_ref[0:1, :] / E
        var = ust_ref[1:2, :] / E - mu * mu
        s2 = g_ref[...] / jnp.sqrt(var + 1e-5)
        t2 = b_ref[...] - mu * s2
        un = jnp.maximum(u_ref[...] * s2 + t2, 0.0)
        v = jnp.dot(un, wd56_ref[...], preferred_element_type=jnp.float32) + bd56_ref[...]
        out_ref[...] = jnp.reshape(1.0 / (1.0 + jnp.exp(-v)),
                                   (BLK // CHUNK, CHUNK))

    tc['h'] = pl.pallas_call(
        h_body,
        grid=(NBLK,),
        in_specs=[pl.BlockSpec((BLK, 16), lambda i: (i, 0)),
                  pl.BlockSpec((2, 16), lambda i: (0, 0)),
                  pl.BlockSpec((1, 16), lambda i: (0, 0)),
                  pl.BlockSpec((1, 16), lambda i: (0, 0)),
                  pl.BlockSpec((16, 1), lambda i: (0, 0)),
                  pl.BlockSpec((1, 1), lambda i: (0, 0))],
        out_specs=pl.BlockSpec((BLK // CHUNK, CHUNK), lambda i: (i, 0)),
        out_shape=jax.ShapeDtypeStruct((EP // CHUNK, CHUNK), jnp.float32),
        interpret=interpret)

    return tc


_IMPL = []


def _get_impl():
    if not _IMPL:
        tc = _build_tc()
        sc = {
            'g32': _make_gather(32, 32),
            's17': _make_scatter(17),
            'gat16': _make_gat(16),
            'gat32': _make_gat(32),
        }
        _IMPL.append((tc, sc))
    return _IMPL[0]


def _pipeline(x, edge_index, edge_attr, params, tc, sc):
    p = params
    src = edge_index[0]
    dst = edge_index[1]
    pad_e = EP - E
    src_p = jnp.pad(src, (0, pad_e)).astype(jnp.int32).reshape(EP // CHUNK, CHUNK)
    dst_p = jnp.pad(dst, (0, pad_e)).astype(jnp.int32).reshape(EP // CHUNK, CHUNK)
    ea_p = jnp.pad(edge_attr, ((0, pad_e), (0, 0)))
    ea1_p = jnp.pad(jnp.concatenate(
        [edge_attr, jnp.ones((E, 1), jnp.float32)], axis=1),
        ((0, pad_e), (0, 0)))
    z17 = jnp.zeros((N2, 17), jnp.float32)
    z1d = jnp.zeros((N2,), jnp.float32)

    wv1 = (p['We1'] @ p['att_edge1']).reshape(16, 1)
    wv2 = (p['We2'] @ p['att_edge2']).reshape(16, 1)
    wv12 = jnp.concatenate([wv1, wv2], axis=1)
    wd12 = p['Wd1'] @ p['Wd2']
    bd12 = (p['bd1'] @ p['Wd2'] + p['bd2']).reshape(1, 32)
    wd34 = p['Wd3'] @ p['Wd4']
    bd34 = (p['bd3'] @ p['Wd4'] + p['bd4']).reshape(1, 16)
    wd56 = p['Wd5'] @ p['Wd6']
    bd56 = (p['bd5'] @ p['Wd6'] + p['bd6']).reshape(1, 1)

    parts0 = sc['s17'](ea1_p, dst_p, z17)

    h1, asrc1, adt1, aself1, ael2 = tc['a1'](
        x, parts0, p['W1'], p['att_src1'].reshape(16, 1),
        p['att_dst1'].reshape(16, 1), wv1, wv2)
    ae1, ae2 = tc['a2'](ea_p, wv12)

    parts1 = sc['gat16'](h1, asrc1.reshape(N), adt1.reshape(N),
                         ae1, src_p, dst_p, z1d)
    agg1 = (parts1[0:17] + parts1[17:34]).T

    h2, asrc2, adt2, aself2 = tc['c'](
        agg1, h1, aself1, ael2, p['b1'].reshape(1, 16),
        p['bn16_g'].reshape(1, 16), p['bn16_b'].reshape(1, 16),
        p['W2'], p['att_src2'].reshape(32, 1), p['att_dst2'].reshape(32, 1))

    parts2 = sc['gat32'](h2, asrc2.reshape(N), adt2.reshape(N),
                         ae2, src_p, dst_p, z1d)
    agg2 = (parts2[0:33] + parts2[33:66]).T
    hout = tc['e'](agg2, h2, aself2, p['b2'].reshape(1, 32))

    gs, gd = sc['g32'](hout, hout, src_p, dst_p)
    z, zst = tc['f'](gs, gd, ea_p,
                     p['Wm1'][0:32], p['Wm1'][32:64], p['Wm1'][64:80],
                     p['bm1'].reshape(1, 32), p['Wm2'], p['bm2'].reshape(1, 32),
                     wd12[0:32], wd12[32:48], bd12)
    u, ust = tc['g'](z, zst, p['bnd32_g'].reshape(1, 32),
                     p['bnd32_b'].reshape(1, 32), wd34, bd34)
    outp = tc['h'](u, ust, p['bnd16_g'].reshape(1, 16),
                   p['bnd16_b'].reshape(1, 16), wd56, bd56)
    return outp.reshape(EP, 1)[:E]


def kernel(x, edge_index, edge_attr, params, P, D, K):
    tc, sc = _get_impl()
    return _pipeline(x, edge_index, edge_attr, params, tc, sc)
